# Initial kernel scaffold; baseline (speedup 1.0000x reference)
import jax
import jax.numpy as jnp
from jax.experimental import pallas as pl


def kernel(positions, senders, receivers, graph_ids, W_pre1, W_post1, W_sc1, W_pre2, W_post2, W_sc2, W_pre3, W_post3, W_sc3):
    raise NotImplementedError("write your pallas kernel here")


# speedup vs baseline: 4.3769x; 4.3769x over previous
import functools
import numpy as np
import jax
import jax.numpy as jnp
from jax import lax
from jax.experimental import pallas as pl
from jax.experimental.pallas import tpu as pltpu
from jax.experimental.pallas import tpu_sc as plsc

N = 10000
E = 160000
NG = 64
HID = 192
BN = 157
NBLK = 64
NPAD = BN * NBLK
TPE = 5024
EPAD = 32 * TPE
CH = 64
WAGG = 448

_mesh = plsc.VectorSubcoreMesh(core_axis_name="c", subcore_axis_name="s")


def _wid():
    return lax.axis_index("s") * 2 + lax.axis_index("c")


def _rsqrt(r2):
    i = plsc.bitcast(r2, jnp.int32)
    i = jnp.int32(0x5F3759DF) - lax.shift_right_logical(i, 1)
    y = plsc.bitcast(i, jnp.float32)
    for _ in range(3):
        y = y * (1.5 - 0.5 * r2 * y * y)
    return y


@functools.partial(
    pl.kernel, mesh=_mesh,
    compiler_params=pltpu.CompilerParams(needs_layout_passes=False),
    out_type=jax.ShapeDtypeStruct((EPAD * 16,), jnp.float32),
    scratch_types=[
        pltpu.VMEM((N,), jnp.float32),
        pltpu.VMEM((N,), jnp.float32),
        pltpu.VMEM((N,), jnp.float32),
        pltpu.VMEM((TPE,), jnp.int32),
        pltpu.VMEM((TPE,), jnp.int32),
        pltpu.VMEM((TPE * 16,), jnp.float32),
    ],
)
def _sh_kernel(px_h, py_h, pz_h, ss_h, sr_h, sh_h, px, py, pz, ssv, srv, shb):
    wid = _wid()
    base = wid * TPE
    pltpu.sync_copy(px_h, px)
    pltpu.sync_copy(py_h, py)
    pltpu.sync_copy(pz_h, pz)
    pltpu.sync_copy(ss_h.at[pl.ds(base, TPE)], ssv)
    pltpu.sync_copy(sr_h.at[pl.ds(base, TPE)], srv)
    lane16 = lax.iota(jnp.int32, 16) * 16
    c3 = np.float32(np.sqrt(3.0))
    c5 = np.float32(np.sqrt(5.0))
    c7h = np.float32(np.sqrt(7.0) * 0.5)
    ones = jnp.ones((16,), jnp.float32)

    def chunk(k, carry):
        sv = ssv[pl.ds(k * 16, 16)]
        rv = srv[pl.ds(k * 16, 16)]
        dx = plsc.load_gather(px, [rv]) - plsc.load_gather(px, [sv])
        dy = plsc.load_gather(py, [rv]) - plsc.load_gather(py, [sv])
        dz = plsc.load_gather(pz, [rv]) - plsc.load_gather(pz, [sv])
        r2 = jnp.maximum(dx * dx + dy * dy + dz * dz, jnp.float32(1e-18))
        rin = _rsqrt(r2)
        ux, uy, uz = dx * rin, dy * rin, dz * rin
        x2, y2, z2 = ux * ux, uy * uy, uz * uz
        vals = (uy * c3, uz * c3, ux * c3,
                ux * uy * c5, uy * uz * c5, (1.5 * z2 - 0.5) * c5,
                ux * uz * c5, 0.5 * (x2 - y2) * c5,
                uy * (3.0 * x2 - y2) * c7h, ux * uy * uz * c7h,
                uy * (5.0 * z2 - 1.0) * c7h, uz * (5.0 * z2 - 3.0) * c7h,
                ux * (5.0 * z2 - 1.0) * c7h, uz * (x2 - y2) * c7h,
                ux * (x2 - 3.0 * y2) * c7h, ones)
        for j, v in enumerate(vals):
            plsc.store_scatter(shb, [lane16 + (k * 256 + j)], v)
        return carry

    lax.fori_loop(0, TPE // 16, chunk, 0)
    pltpu.sync_copy(shb, sh_h.at[pl.ds(base * 16, TPE * 16)])


@functools.partial(
    pl.kernel, mesh=_mesh,
    compiler_params=pltpu.CompilerParams(needs_layout_passes=False),
    out_type=jax.ShapeDtypeStruct((NPAD * 16,), jnp.float32),
    scratch_types=[
        pltpu.VMEM((16,), jnp.int32),
        pltpu.VMEM((BN * 16,), jnp.float32),
        pltpu.VMEM((CH,), jnp.int32),
        pltpu.VMEM((CH * 16,), jnp.float32),
    ],
)
def _agg1_kernel(shf_h, sr_h, bo_h, agg_h, bov, acc, srv, shg):
    wid = _wid()
    pltpu.sync_copy(bo_h.at[wid], bov)
    bvec = bov[...]
    zero16 = jnp.zeros((16,), jnp.float32)
    for bi in range(2):
        blk = wid * 2 + bi

        def zr(i, carry):
            acc[pl.ds(i * 16, 16)] = zero16
            return carry
        lax.fori_loop(0, BN, zr, 0)

        e0 = bvec[bi]
        e1 = bvec[bi + 1]
        astart = (e0 // 8) * 8
        nch = (e1 - astart + CH - 1) // CH
        nbase = blk * BN

        def chunk(c, carry):
            cbase = astart + c * CH
            pltpu.sync_copy(sr_h.at[pl.ds(cbase, CH)], srv)
            pltpu.sync_copy(shf_h.at[pl.ds(cbase * 16, CH * 16)], shg)

            def group(g, carry2):
                gb = g * 16
                rv16 = srv[pl.ds(gb, 16)]
                for es in range(16):
                    e = gb + es
                    ge = cbase + e

                    @pl.when((ge >= e0) & (ge < e1))
                    def _():
                        row = rv16[es] - nbase
                        plsc.addupdate(acc.at[pl.ds(row * 16, 16)],
                                       shg[pl.ds(e * 16, 16)])
                return carry2
            lax.fori_loop(0, CH // 16, group, 0)
            return carry
        lax.fori_loop(0, nch, chunk, 0)
        pltpu.sync_copy(acc, agg_h.at[pl.ds(blk * BN * 16, BN * 16)])


@functools.partial(
    pl.kernel, mesh=_mesh,
    compiler_params=pltpu.CompilerParams(needs_layout_passes=False),
    out_type=jax.ShapeDtypeStruct((NPAD * WAGG,), jnp.float32),
    scratch_types=[
        pltpu.VMEM((16,), jnp.int32),
        pltpu.VMEM((BN * WAGG,), jnp.float32),
        pltpu.VMEM((CH,), jnp.int32),
        pltpu.VMEM((CH,), jnp.int32),
        pltpu.VMEM((CH, 256), jnp.float32),
        pltpu.VMEM((CH * 16,), jnp.float32),
        pltpu.SemaphoreType.DMA,
    ],
)
def _agg_kernel(x_h, shf_h, ss_h, sr_h, bo_h, agg_h,
                bov, acc, ssv, srv, xg, shg, sem):
    wid = _wid()
    pltpu.sync_copy(bo_h.at[wid], bov)
    bvec = bov[...]
    zero16 = jnp.zeros((16,), jnp.float32)
    for bi in range(2):
        blk = wid * 2 + bi

        def zr(i, carry):
            acc[pl.ds(i * 16, 16)] = zero16
            return carry
        lax.fori_loop(0, BN * WAGG // 16, zr, 0)

        e0 = bvec[bi]
        e1 = bvec[bi + 1]
        astart = (e0 // 8) * 8
        nch = (e1 - astart + CH - 1) // CH
        nbase = blk * BN

        def chunk(c, carry):
            cbase = astart + c * CH
            pltpu.sync_copy(ss_h.at[pl.ds(cbase, CH)], ssv)
            pltpu.sync_copy(sr_h.at[pl.ds(cbase, CH)], srv)
            cp = pltpu.async_copy(x_h.at[ssv], xg, sem)
            pltpu.sync_copy(shf_h.at[pl.ds(cbase * 16, CH * 16)], shg)
            cp.wait()

            def group(g, carry2):
                gb = g * 16
                rv16 = srv[pl.ds(gb, 16)]
                for es in range(16):
                    e = gb + es
                    ge = cbase + e

                    @pl.when((ge >= e0) & (ge < e1))
                    def _():
                        row = rv16[es] - nbase
                        rb = row * WAGG
                        shv = shg[pl.ds(e * 16, 16)]
                        x16 = xg[e, pl.ds(0, 16)]
                        for k in range(12):
                            plsc.addupdate(acc.at[pl.ds(rb + k * 16, 16)],
                                           xg[e, pl.ds(k * 16, 16)])
                        for i in range(16):
                            b = jnp.broadcast_to(x16[i], (16,))
                            plsc.addupdate(
                                acc.at[pl.ds(rb + HID + i * 16, 16)],
                                b * shv)
                return carry2
            lax.fori_loop(0, CH // 16, group, 0)
            return carry
        lax.fori_loop(0, nch, chunk, 0)
        pltpu.sync_copy(acc, agg_h.at[pl.ds(blk * BN * WAGG, BN * WAGG)])


def _t1_body(agg_ref, wp_ref, wo_ref, wsc_ref, out_ref):
    h = jax.nn.gelu(jnp.dot(agg_ref[...] / 1.5, wp_ref[...],
                            preferred_element_type=jnp.float32))
    h = jnp.dot(h, wo_ref[...], preferred_element_type=jnp.float32)
    h = h + wsc_ref[...]
    out_ref[...] = jnp.concatenate(
        [h, jnp.zeros((h.shape[0], 256 - HID), jnp.float32)], axis=1)


def _t2_body(agg_ref, x_ref, wp_ref, wo_ref, wsc_ref, out_ref):
    h = jax.nn.gelu(jnp.dot(agg_ref[...] / 1.5, wp_ref[...],
                            preferred_element_type=jnp.float32))
    h = jnp.dot(h, wo_ref[...], preferred_element_type=jnp.float32)
    h = h + jnp.dot(x_ref[...], wsc_ref[...],
                    preferred_element_type=jnp.float32)
    out_ref[...] = jnp.concatenate(
        [h, jnp.zeros((h.shape[0], 256 - HID), jnp.float32)], axis=1)


def _t3_body(agg_ref, x_ref, gi_ref, wp_ref, wo_ref, wsc_ref, out_ref, pred):
    g = pl.program_id(0)

    @pl.when(g == 0)
    def _():
        pred[...] = jnp.zeros_like(pred)

    h = jax.nn.gelu(jnp.dot(agg_ref[...] / 1.5, wp_ref[...],
                            preferred_element_type=jnp.float32))
    h = jnp.dot(h, wo_ref[...], preferred_element_type=jnp.float32)
    x3 = h + jnp.dot(x_ref[...], wsc_ref[...],
                     preferred_element_type=jnp.float32)
    gi = gi_ref[0, 0, :]
    oh = (lax.broadcasted_iota(jnp.int32, (NG, 1000), 0)
          == gi[None, :]).astype(jnp.float32)
    pred[...] += jnp.dot(oh, x3, preferred_element_type=jnp.float32)

    @pl.when(g == 9)
    def _():
        p = pred[...]
        oe = p[:, 0:1] * p[:, 1:2]
        out_ref[...] = jnp.concatenate([oe, -oe, p[:, 2:8]], axis=1)


def _full(i, j):
    return pl.BlockSpec(j, lambda g: tuple(0 for _ in j)) if i is None else None


def _reshuffle(Wp):
    a = Wp[:HID]
    b = Wp[HID:].reshape(16, 15, -1)
    b = jnp.pad(b, ((0, 0), (0, 1), (0, 0)))
    return jnp.concatenate([a, b.reshape(256, -1)], axis=0)


def kernel(positions, senders, receivers, graph_ids,
           W_pre1, W_post1, W_sc1, W_pre2, W_post2, W_sc2,
           W_pre3, W_post3, W_sc3):
    senders = senders.astype(jnp.int32)
    receivers = receivers.astype(jnp.int32)
    graph_ids = graph_ids.astype(jnp.int32)
    order = jnp.argsort(receivers)
    ss = senders[order]
    sr = receivers[order]
    ss_p = jnp.pad(ss, (0, EPAD - E))
    sr_p = jnp.pad(sr, (0, EPAD - E))
    bo = jnp.searchsorted(sr, jnp.arange(NBLK + 1, dtype=jnp.int32) * BN)
    bo = bo.astype(jnp.int32)
    bo2 = jnp.stack([bo[0:64:2], bo[1:64:2], bo[2:65:2]], axis=1)
    bo2 = jnp.pad(bo2, ((0, 0), (0, 13)))
    px = positions[:, 0]
    py = positions[:, 1]
    pz = positions[:, 2]

    shf = _sh_kernel(px, py, pz, ss_p, sr_p)
    agg1 = _agg1_kernel(shf, sr_p, bo2).reshape(NPAD, 16)

    Wp1p = jnp.concatenate([W_pre1[1:16], W_pre1[0:1]], axis=0)
    x1 = pl.pallas_call(
        _t1_body, grid=(10,),
        in_specs=[pl.BlockSpec((1000, 16), lambda g: (g, 0)),
                  pl.BlockSpec((16, HID), lambda g: (0, 0)),
                  pl.BlockSpec((HID, HID), lambda g: (0, 0)),
                  pl.BlockSpec((1, HID), lambda g: (0, 0))],
        out_specs=pl.BlockSpec((1000, 256), lambda g: (g, 0)),
        out_shape=jax.ShapeDtypeStruct((N, 256), jnp.float32),
    )(agg1, Wp1p, W_post1, W_sc1)

    agg2 = _agg_kernel(x1, shf, ss_p, sr_p, bo2).reshape(NPAD, WAGG)
    x2 = pl.pallas_call(
        _t2_body, grid=(10,),
        in_specs=[pl.BlockSpec((1000, WAGG), lambda g: (g, 0)),
                  pl.BlockSpec((1000, 256), lambda g: (g, 0)),
                  pl.BlockSpec((WAGG, HID), lambda g: (0, 0)),
                  pl.BlockSpec((HID, HID), lambda g: (0, 0)),
                  pl.BlockSpec((256, HID), lambda g: (0, 0))],
        out_specs=pl.BlockSpec((1000, 256), lambda g: (g, 0)),
        out_shape=jax.ShapeDtypeStruct((N, 256), jnp.float32),
    )(agg2, x1, _reshuffle(W_pre2), W_post2,
      jnp.pad(W_sc2, ((0, 256 - HID), (0, 0))))

    agg3 = _agg_kernel(x2, shf, ss_p, sr_p, bo2).reshape(NPAD, WAGG)
    gi3 = graph_ids.reshape(10, 1, 1000)
    logits = pl.pallas_call(
        _t3_body, grid=(10,),
        in_specs=[pl.BlockSpec((1000, WAGG), lambda g: (g, 0)),
                  pl.BlockSpec((1000, 256), lambda g: (g, 0)),
                  pl.BlockSpec((1, 1, 1000), lambda g: (g, 0, 0)),
                  pl.BlockSpec((WAGG, 8), lambda g: (0, 0)),
                  pl.BlockSpec((8, 8), lambda g: (0, 0)),
                  pl.BlockSpec((256, 8), lambda g: (0, 0))],
        out_specs=pl.BlockSpec((NG, 8), lambda g: (0, 0)),
        out_shape=jax.ShapeDtypeStruct((NG, 8), jnp.float32),
        scratch_shapes=[pltpu.VMEM((NG, 8), jnp.float32)],
    )(agg3, x2, gi3, _reshuffle(W_pre3), W_post3,
      jnp.pad(W_sc3, ((0, 256 - HID), (0, 0))))
    return logits


# speedup vs baseline: 5.0174x; 1.1463x over previous
import functools
import numpy as np
import jax
import jax.numpy as jnp
from jax import lax
from jax.experimental import pallas as pl
from jax.experimental.pallas import tpu as pltpu
from jax.experimental.pallas import tpu_sc as plsc

N = 10000
E = 160000
NG = 64
HID = 192
BN = 157
NBLK = 64
NPAD = BN * NBLK
TPE = 5024
EPAD = 32 * TPE
CH = 64
WAGG = 448

_mesh = plsc.VectorSubcoreMesh(core_axis_name="c", subcore_axis_name="s")


def _wid():
    return lax.axis_index("s") * 2 + lax.axis_index("c")


def _rsqrt(r2):
    i = plsc.bitcast(r2, jnp.int32)
    i = jnp.int32(0x5F3759DF) - lax.shift_right_logical(i, 1)
    y = plsc.bitcast(i, jnp.float32)
    for _ in range(3):
        y = y * (1.5 - 0.5 * r2 * y * y)
    return y


@functools.partial(
    pl.kernel, mesh=_mesh,
    compiler_params=pltpu.CompilerParams(needs_layout_passes=False),
    out_type=jax.ShapeDtypeStruct((EPAD * 16,), jnp.float32),
    scratch_types=[
        pltpu.VMEM((N,), jnp.float32),
        pltpu.VMEM((N,), jnp.float32),
        pltpu.VMEM((N,), jnp.float32),
        pltpu.VMEM((TPE,), jnp.int32),
        pltpu.VMEM((TPE,), jnp.int32),
        pltpu.VMEM((TPE * 16,), jnp.float32),
    ],
)
def _sh_kernel(px_h, py_h, pz_h, ss_h, sr_h, sh_h, px, py, pz, ssv, srv, shb):
    wid = _wid()
    base = wid * TPE
    pltpu.sync_copy(px_h, px)
    pltpu.sync_copy(py_h, py)
    pltpu.sync_copy(pz_h, pz)
    pltpu.sync_copy(ss_h.at[pl.ds(base, TPE)], ssv)
    pltpu.sync_copy(sr_h.at[pl.ds(base, TPE)], srv)
    lane16 = lax.iota(jnp.int32, 16) * 16
    c3 = np.float32(np.sqrt(3.0))
    c5 = np.float32(np.sqrt(5.0))
    c7h = np.float32(np.sqrt(7.0) * 0.5)
    ones = jnp.ones((16,), jnp.float32)

    def chunk(k, carry):
        sv = ssv[pl.ds(k * 16, 16)]
        rv = srv[pl.ds(k * 16, 16)]
        dx = plsc.load_gather(px, [rv]) - plsc.load_gather(px, [sv])
        dy = plsc.load_gather(py, [rv]) - plsc.load_gather(py, [sv])
        dz = plsc.load_gather(pz, [rv]) - plsc.load_gather(pz, [sv])
        r2 = jnp.maximum(dx * dx + dy * dy + dz * dz, jnp.float32(1e-18))
        rin = _rsqrt(r2)
        ux, uy, uz = dx * rin, dy * rin, dz * rin
        x2, y2, z2 = ux * ux, uy * uy, uz * uz
        vals = (uy * c3, uz * c3, ux * c3,
                ux * uy * c5, uy * uz * c5, (1.5 * z2 - 0.5) * c5,
                ux * uz * c5, 0.5 * (x2 - y2) * c5,
                uy * (3.0 * x2 - y2) * c7h, ux * uy * uz * c7h,
                uy * (5.0 * z2 - 1.0) * c7h, uz * (5.0 * z2 - 3.0) * c7h,
                ux * (5.0 * z2 - 1.0) * c7h, uz * (x2 - y2) * c7h,
                ux * (x2 - 3.0 * y2) * c7h, ones)
        for j, v in enumerate(vals):
            plsc.store_scatter(shb, [lane16 + (k * 256 + j)], v)
        return carry

    lax.fori_loop(0, TPE // 16, chunk, 0)
    pltpu.sync_copy(shb, sh_h.at[pl.ds(base * 16, TPE * 16)])


@functools.partial(
    pl.kernel, mesh=_mesh,
    compiler_params=pltpu.CompilerParams(needs_layout_passes=False),
    out_type=jax.ShapeDtypeStruct((NPAD * 16,), jnp.float32),
    scratch_types=[
        pltpu.VMEM((16,), jnp.int32),
        pltpu.VMEM((BN * 16,), jnp.float32),
        pltpu.VMEM((CH,), jnp.int32),
        pltpu.VMEM((CH * 16,), jnp.float32),
    ],
)
def _agg1_kernel(shf_h, sr_h, bo_h, agg_h, bov, acc, srv, shg):
    wid = _wid()
    pltpu.sync_copy(bo_h.at[wid], bov)
    bvec = bov[...]
    zero16 = jnp.zeros((16,), jnp.float32)
    for bi in range(2):
        blk = wid * 2 + bi

        def zr(i, carry):
            acc[pl.ds(i * 16, 16)] = zero16
            return carry
        lax.fori_loop(0, BN, zr, 0)

        e0 = bvec[bi]
        e1 = bvec[bi + 1]
        astart = (e0 // 8) * 8
        nch = (e1 - astart + CH - 1) // CH
        nbase = blk * BN

        def chunk(c, carry):
            cbase = astart + c * CH
            pltpu.sync_copy(sr_h.at[pl.ds(cbase, CH)], srv)
            pltpu.sync_copy(shf_h.at[pl.ds(cbase * 16, CH * 16)], shg)

            def group(g, carry2):
                gb = g * 16
                rv16 = srv[pl.ds(gb, 16)]
                for es in range(16):
                    e = gb + es
                    ge = cbase + e

                    @pl.when((ge >= e0) & (ge < e1))
                    def _():
                        row = rv16[es] - nbase
                        plsc.addupdate(acc.at[pl.ds(row * 16, 16)],
                                       shg[pl.ds(e * 16, 16)])
                return carry2
            lax.fori_loop(0, CH // 16, group, 0)
            return carry
        lax.fori_loop(0, nch, chunk, 0)
        pltpu.sync_copy(acc, agg_h.at[pl.ds(blk * BN * 16, BN * 16)])


@functools.partial(
    pl.kernel, mesh=_mesh,
    compiler_params=pltpu.CompilerParams(needs_layout_passes=False),
    out_type=jax.ShapeDtypeStruct((NPAD * WAGG,), jnp.float32),
    scratch_types=[
        pltpu.VMEM((16,), jnp.int32),
        pltpu.VMEM((BN * WAGG,), jnp.float32),
        pltpu.VMEM((CH,), jnp.int32),
        pltpu.VMEM((CH,), jnp.int32),
        pltpu.VMEM((CH * 2,), jnp.int32),
        pltpu.VMEM((CH * 2,), jnp.int32),
        pltpu.VMEM((CH, 256), jnp.float32),
        pltpu.VMEM((CH, 256), jnp.float32),
        pltpu.VMEM((CH * 16,), jnp.float32),
        pltpu.VMEM((CH * 16,), jnp.float32),
        pltpu.SemaphoreType.DMA,
        pltpu.SemaphoreType.DMA,
        pltpu.SemaphoreType.DMA,
        pltpu.SemaphoreType.DMA,
    ],
)
def _agg_kernel(x_h, shf_h, ids_h, bo_h, agg_h,
                bov, acc, ssv0, ssv1, idsb0, idsb1, xg0, xg1, shg0, shg1,
                sem_s0, sem_s1, sem_x0, sem_x1):
    wid = _wid()
    pltpu.sync_copy(bo_h.at[wid], bov)
    bvec = bov[...]
    zero16 = jnp.zeros((16,), jnp.float32)
    lane2 = lax.iota(jnp.int32, 16) * 2
    idsb = (idsb0, idsb1)
    ssv = (ssv0, ssv1)
    xg = (xg0, xg1)
    shg = (shg0, shg1)
    sem_s = (sem_s0, sem_s1)
    sem_x = (sem_x0, sem_x1)
    for bi in range(2):
        blk = wid * 2 + bi

        def zr(i, carry):
            acc[pl.ds(i * 16, 16)] = zero16
            return carry
        lax.fori_loop(0, BN * WAGG // 16, zr, 0)

        e0 = bvec[bi]
        e1 = bvec[bi + 1]
        astart = (e0 // 8) * 8
        nch = (e1 - astart + CH - 1) // CH
        nbase = blk * BN

        def issue_shx(c, s):
            @pl.when(c < nch)
            def _():
                cbase = astart + c * CH
                pltpu.async_copy(ids_h.at[pl.ds(cbase * 2, CH * 2)],
                                 idsb[s], sem_s[s])
                pltpu.async_copy(shf_h.at[pl.ds(cbase * 16, CH * 16)],
                                 shg[s], sem_s[s])

        def finish_shx_issue_gather(c, s):
            @pl.when(c < nch)
            def _():
                cbase = astart + c * CH
                pltpu.make_async_copy(ids_h.at[pl.ds(cbase * 2, CH * 2)],
                                      idsb[s], sem_s[s]).wait()
                pltpu.make_async_copy(shf_h.at[pl.ds(cbase * 16, CH * 16)],
                                      shg[s], sem_s[s]).wait()
                for g in range(CH // 16):
                    sv = plsc.load_gather(idsb[s], [lane2 + (g * 32)])
                    ssv[s][pl.ds(g * 16, 16)] = sv
                pltpu.async_copy(x_h.at[ssv[s]], xg[s], sem_x[s])

        def process(c, s):
            @pl.when(c < nch)
            def _():
                cbase = astart + c * CH
                pltpu.make_async_copy(x_h.at[ssv[s]], xg[s], sem_x[s]).wait()

                def group(g, carry2):
                    gb = g * 16
                    rv16 = plsc.load_gather(idsb[s], [lane2 + (gb * 2 + 1)])
                    for es in range(16):
                        e = gb + es
                        ge = cbase + e

                        @pl.when((ge >= e0) & (ge < e1))
                        def _():
                            row = rv16[es] - nbase
                            rb = row * WAGG
                            shv = shg[s][pl.ds(e * 16, 16)]
                            x16 = xg[s][e, pl.ds(0, 16)]
                            for k in range(12):
                                plsc.addupdate(
                                    acc.at[pl.ds(rb + k * 16, 16)],
                                    xg[s][e, pl.ds(k * 16, 16)])
                            for i in range(16):
                                b = jnp.broadcast_to(x16[i], (16,))
                                plsc.addupdate(
                                    acc.at[pl.ds(rb + HID + i * 16, 16)],
                                    b * shv)
                    return carry2
                lax.fori_loop(0, CH // 16, group, 0)

        issue_shx(0, 0)
        finish_shx_issue_gather(0, 0)
        issue_shx(1, 1)

        def pair(p, carry):
            c0 = 2 * p
            c1 = c0 + 1
            finish_shx_issue_gather(c1, 1)
            process(c0, 0)
            issue_shx(c0 + 2, 0)
            process(c1, 1)
            finish_shx_issue_gather(c0 + 2, 0)
            issue_shx(c1 + 2, 1)
            return carry
        lax.fori_loop(0, (nch + 1) // 2, pair, 0)
        pltpu.sync_copy(acc, agg_h.at[pl.ds(blk * BN * WAGG, BN * WAGG)])


def _t1_body(agg_ref, wp_ref, wo_ref, wsc_ref, out_ref):
    h = jax.nn.gelu(jnp.dot(agg_ref[...] / 1.5, wp_ref[...],
                            preferred_element_type=jnp.float32))
    h = jnp.dot(h, wo_ref[...], preferred_element_type=jnp.float32)
    h = h + wsc_ref[...]
    out_ref[...] = jnp.concatenate(
        [h, jnp.zeros((h.shape[0], 256 - HID), jnp.float32)], axis=1)


def _t2_body(agg_ref, x_ref, wp_ref, wo_ref, wsc_ref, out_ref):
    h = jax.nn.gelu(jnp.dot(agg_ref[...] / 1.5, wp_ref[...],
                            preferred_element_type=jnp.float32))
    h = jnp.dot(h, wo_ref[...], preferred_element_type=jnp.float32)
    h = h + jnp.dot(x_ref[...], wsc_ref[...],
                    preferred_element_type=jnp.float32)
    out_ref[...] = jnp.concatenate(
        [h, jnp.zeros((h.shape[0], 256 - HID), jnp.float32)], axis=1)


def _t3_body(agg_ref, x_ref, gi_ref, wp_ref, wo_ref, wsc_ref, out_ref, pred):
    g = pl.program_id(0)

    @pl.when(g == 0)
    def _():
        pred[...] = jnp.zeros_like(pred)

    h = jax.nn.gelu(jnp.dot(agg_ref[...] / 1.5, wp_ref[...],
                            preferred_element_type=jnp.float32))
    h = jnp.dot(h, wo_ref[...], preferred_element_type=jnp.float32)
    x3 = h + jnp.dot(x_ref[...], wsc_ref[...],
                     preferred_element_type=jnp.float32)
    gi = gi_ref[0, 0, :]
    oh = (lax.broadcasted_iota(jnp.int32, (NG, 1000), 0)
          == gi[None, :]).astype(jnp.float32)
    pred[...] += jnp.dot(oh, x3, preferred_element_type=jnp.float32)

    @pl.when(g == 9)
    def _():
        p = pred[...]
        oe = p[:, 0:1] * p[:, 1:2]
        out_ref[...] = jnp.concatenate([oe, -oe, p[:, 2:8]], axis=1)


def _full(i, j):
    return pl.BlockSpec(j, lambda g: tuple(0 for _ in j)) if i is None else None


def _reshuffle(Wp):
    a = Wp[:HID]
    b = Wp[HID:].reshape(16, 15, -1)
    b = jnp.pad(b, ((0, 0), (0, 1), (0, 0)))
    return jnp.concatenate([a, b.reshape(256, -1)], axis=0)


def kernel(positions, senders, receivers, graph_ids,
           W_pre1, W_post1, W_sc1, W_pre2, W_post2, W_sc2,
           W_pre3, W_post3, W_sc3):
    senders = senders.astype(jnp.int32)
    receivers = receivers.astype(jnp.int32)
    graph_ids = graph_ids.astype(jnp.int32)
    order = jnp.argsort(receivers)
    ss = senders[order]
    sr = receivers[order]
    ss_p = jnp.pad(ss, (0, EPAD - E))
    sr_p = jnp.pad(sr, (0, EPAD - E))
    ids_p = jnp.stack([ss_p, sr_p], axis=1).reshape(-1)
    bo = jnp.searchsorted(sr, jnp.arange(NBLK + 1, dtype=jnp.int32) * BN)
    bo = bo.astype(jnp.int32)
    bo2 = jnp.stack([bo[0:64:2], bo[1:64:2], bo[2:65:2]], axis=1)
    bo2 = jnp.pad(bo2, ((0, 0), (0, 13)))
    px = positions[:, 0]
    py = positions[:, 1]
    pz = positions[:, 2]

    shf = _sh_kernel(px, py, pz, ss_p, sr_p)
    agg1 = _agg1_kernel(shf, sr_p, bo2).reshape(NPAD, 16)

    Wp1p = jnp.concatenate([W_pre1[1:16], W_pre1[0:1]], axis=0)
    x1 = pl.pallas_call(
        _t1_body, grid=(10,),
        in_specs=[pl.BlockSpec((1000, 16), lambda g: (g, 0)),
                  pl.BlockSpec((16, HID), lambda g: (0, 0)),
                  pl.BlockSpec((HID, HID), lambda g: (0, 0)),
                  pl.BlockSpec((1, HID), lambda g: (0, 0))],
        out_specs=pl.BlockSpec((1000, 256), lambda g: (g, 0)),
        out_shape=jax.ShapeDtypeStruct((N, 256), jnp.float32),
    )(agg1, Wp1p, W_post1, W_sc1)

    agg2 = _agg_kernel(x1, shf, ids_p, bo2).reshape(NPAD, WAGG)
    x2 = pl.pallas_call(
        _t2_body, grid=(10,),
        in_specs=[pl.BlockSpec((1000, WAGG), lambda g: (g, 0)),
                  pl.BlockSpec((1000, 256), lambda g: (g, 0)),
                  pl.BlockSpec((WAGG, HID), lambda g: (0, 0)),
                  pl.BlockSpec((HID, HID), lambda g: (0, 0)),
                  pl.BlockSpec((256, HID), lambda g: (0, 0))],
        out_specs=pl.BlockSpec((1000, 256), lambda g: (g, 0)),
        out_shape=jax.ShapeDtypeStruct((N, 256), jnp.float32),
    )(agg2, x1, _reshuffle(W_pre2), W_post2,
      jnp.pad(W_sc2, ((0, 256 - HID), (0, 0))))

    agg3 = _agg_kernel(x2, shf, ids_p, bo2).reshape(NPAD, WAGG)
    gi3 = graph_ids.reshape(10, 1, 1000)
    logits = pl.pallas_call(
        _t3_body, grid=(10,),
        in_specs=[pl.BlockSpec((1000, WAGG), lambda g: (g, 0)),
                  pl.BlockSpec((1000, 256), lambda g: (g, 0)),
                  pl.BlockSpec((1, 1, 1000), lambda g: (g, 0, 0)),
                  pl.BlockSpec((WAGG, 8), lambda g: (0, 0)),
                  pl.BlockSpec((8, 8), lambda g: (0, 0)),
                  pl.BlockSpec((256, 8), lambda g: (0, 0))],
        out_specs=pl.BlockSpec((NG, 8), lambda g: (0, 0)),
        out_shape=jax.ShapeDtypeStruct((NG, 8), jnp.float32),
        scratch_shapes=[pltpu.VMEM((NG, 8), jnp.float32)],
    )(agg3, x2, gi3, _reshuffle(W_pre3), W_post3,
      jnp.pad(W_sc3, ((0, 256 - HID), (0, 0))))
    return logits


# speedup vs baseline: 6.7285x; 1.3410x over previous
import functools
import numpy as np
import jax
import jax.numpy as jnp
from jax import lax
from jax.experimental import pallas as pl
from jax.experimental.pallas import tpu as pltpu
from jax.experimental.pallas import tpu_sc as plsc

N = 10000
E = 160000
NG = 64
HID = 192
BN = 157
NBLK = 64
NPAD = BN * NBLK
TPE = 5024
EPAD = 32 * TPE
CH = 64
WAGG = 448

_mesh = plsc.VectorSubcoreMesh(core_axis_name="c", subcore_axis_name="s")


def _wid():
    return lax.axis_index("s") * 2 + lax.axis_index("c")


def _rsqrt(r2):
    i = plsc.bitcast(r2, jnp.int32)
    i = jnp.int32(0x5F3759DF) - lax.shift_right_logical(i, 1)
    y = plsc.bitcast(i, jnp.float32)
    for _ in range(3):
        y = y * (1.5 - 0.5 * r2 * y * y)
    return y


@functools.partial(
    pl.kernel, mesh=_mesh,
    compiler_params=pltpu.CompilerParams(needs_layout_passes=False),
    out_type=jax.ShapeDtypeStruct((EPAD * 16,), jnp.float32),
    scratch_types=[
        pltpu.VMEM((N,), jnp.float32),
        pltpu.VMEM((N,), jnp.float32),
        pltpu.VMEM((N,), jnp.float32),
        pltpu.VMEM((TPE,), jnp.int32),
        pltpu.VMEM((TPE,), jnp.int32),
        pltpu.VMEM((TPE * 16,), jnp.float32),
    ],
)
def _sh_kernel(px_h, py_h, pz_h, ss_h, sr_h, sh_h, px, py, pz, ssv, srv, shb):
    wid = _wid()
    base = wid * TPE
    pltpu.sync_copy(px_h, px)
    pltpu.sync_copy(py_h, py)
    pltpu.sync_copy(pz_h, pz)
    pltpu.sync_copy(ss_h.at[pl.ds(base, TPE)], ssv)
    pltpu.sync_copy(sr_h.at[pl.ds(base, TPE)], srv)
    lane16 = lax.iota(jnp.int32, 16) * 16
    c3 = np.float32(np.sqrt(3.0))
    c5 = np.float32(np.sqrt(5.0))
    c7h = np.float32(np.sqrt(7.0) * 0.5)
    ones = jnp.ones((16,), jnp.float32)

    def chunk(k, carry):
        sv = ssv[pl.ds(k * 16, 16)]
        rv = srv[pl.ds(k * 16, 16)]
        dx = plsc.load_gather(px, [rv]) - plsc.load_gather(px, [sv])
        dy = plsc.load_gather(py, [rv]) - plsc.load_gather(py, [sv])
        dz = plsc.load_gather(pz, [rv]) - plsc.load_gather(pz, [sv])
        r2 = jnp.maximum(dx * dx + dy * dy + dz * dz, jnp.float32(1e-18))
        rin = _rsqrt(r2)
        ux, uy, uz = dx * rin, dy * rin, dz * rin
        x2, y2, z2 = ux * ux, uy * uy, uz * uz
        vals = (uy * c3, uz * c3, ux * c3,
                ux * uy * c5, uy * uz * c5, (1.5 * z2 - 0.5) * c5,
                ux * uz * c5, 0.5 * (x2 - y2) * c5,
                uy * (3.0 * x2 - y2) * c7h, ux * uy * uz * c7h,
                uy * (5.0 * z2 - 1.0) * c7h, uz * (5.0 * z2 - 3.0) * c7h,
                ux * (5.0 * z2 - 1.0) * c7h, uz * (x2 - y2) * c7h,
                ux * (x2 - 3.0 * y2) * c7h, ones)
        for j, v in enumerate(vals):
            plsc.store_scatter(shb, [lane16 + (k * 256 + j)], v)
        return carry

    lax.fori_loop(0, TPE // 16, chunk, 0)
    pltpu.sync_copy(shb, sh_h.at[pl.ds(base * 16, TPE * 16)])


@functools.partial(
    pl.kernel, mesh=_mesh,
    compiler_params=pltpu.CompilerParams(needs_layout_passes=False),
    out_type=jax.ShapeDtypeStruct((NPAD * 16,), jnp.float32),
    scratch_types=[
        pltpu.VMEM((16,), jnp.int32),
        pltpu.VMEM((BN * 16,), jnp.float32),
        pltpu.VMEM((CH,), jnp.int32),
        pltpu.VMEM((CH * 16,), jnp.float32),
    ],
)
def _agg1_kernel(shf_h, sr_h, bo_h, agg_h, bov, acc, srv, shg):
    wid = _wid()
    pltpu.sync_copy(bo_h.at[wid], bov)
    bvec = bov[...]
    zero16 = jnp.zeros((16,), jnp.float32)
    for bi in range(2):
        blk = wid * 2 + bi

        def zr(i, carry):
            acc[pl.ds(i * 16, 16)] = zero16
            return carry
        lax.fori_loop(0, BN, zr, 0)

        e0 = bvec[bi]
        e1 = bvec[bi + 1]
        astart = (e0 // 8) * 8
        nch = (e1 - astart + CH - 1) // CH
        nbase = blk * BN

        def chunk(c, carry):
            cbase = astart + c * CH
            pltpu.sync_copy(sr_h.at[pl.ds(cbase, CH)], srv)
            pltpu.sync_copy(shf_h.at[pl.ds(cbase * 16, CH * 16)], shg)

            def group(g, carry2):
                gb = g * 16
                rv16 = srv[pl.ds(gb, 16)]
                for es in range(16):
                    e = gb + es
                    ge = cbase + e

                    @pl.when((ge >= e0) & (ge < e1))
                    def _():
                        row = rv16[es] - nbase
                        plsc.addupdate(acc.at[pl.ds(row * 16, 16)],
                                       shg[pl.ds(e * 16, 16)])
                return carry2
            lax.fori_loop(0, CH // 16, group, 0)
            return carry
        lax.fori_loop(0, nch, chunk, 0)
        pltpu.sync_copy(acc, agg_h.at[pl.ds(blk * BN * 16, BN * 16)])


@functools.partial(
    pl.kernel, mesh=_mesh,
    compiler_params=pltpu.CompilerParams(needs_layout_passes=False),
    out_type=jax.ShapeDtypeStruct((NPAD * WAGG,), jnp.float32),
    scratch_types=[
        pltpu.VMEM((16,), jnp.int32),
        pltpu.VMEM((BN * WAGG,), jnp.float32),
        pltpu.VMEM((CH,), jnp.int32),
        pltpu.VMEM((CH,), jnp.int32),
        pltpu.VMEM((CH * 2,), jnp.int32),
        pltpu.VMEM((CH * 2,), jnp.int32),
        pltpu.VMEM((CH, 256), jnp.float32),
        pltpu.VMEM((CH, 256), jnp.float32),
        pltpu.VMEM((CH * 16,), jnp.float32),
        pltpu.VMEM((CH * 16,), jnp.float32),
        pltpu.SemaphoreType.DMA,
        pltpu.SemaphoreType.DMA,
        pltpu.SemaphoreType.DMA,
        pltpu.SemaphoreType.DMA,
    ],
)
def _agg_kernel(x_h, shf_h, ids_h, bo_h, agg_h,
                bov, acc, ssv0, ssv1, idsb0, idsb1, xg0, xg1, shg0, shg1,
                sem_s0, sem_s1, sem_x0, sem_x1):
    wid = _wid()
    pltpu.sync_copy(bo_h.at[wid], bov)
    bvec = bov[...]
    zero16 = jnp.zeros((16,), jnp.float32)
    lane2 = lax.iota(jnp.int32, 16) * 2
    idsb = (idsb0, idsb1)
    ssv = (ssv0, ssv1)
    xg = (xg0, xg1)
    shg = (shg0, shg1)
    sem_s = (sem_s0, sem_s1)
    sem_x = (sem_x0, sem_x1)

    def block(bi, carry0):
        blk = wid * 2 + bi

        def zr(i, carry):
            acc[pl.ds(i * 16, 16)] = zero16
            return carry
        lax.fori_loop(0, BN * WAGG // 16, zr, 0)

        is0 = bi == 0
        e0 = jnp.where(is0, bvec[0], bvec[1])
        e1 = jnp.where(is0, bvec[1], bvec[2])
        astart = (e0 // 8) * 8
        nch = (e1 - astart + CH - 1) // CH
        nbase = blk * BN

        def issue_shx(c, s):
            @pl.when(c < nch)
            def _():
                cbase = astart + c * CH
                pltpu.async_copy(ids_h.at[pl.ds(cbase * 2, CH * 2)],
                                 idsb[s], sem_s[s])
                pltpu.async_copy(shf_h.at[pl.ds(cbase * 16, CH * 16)],
                                 shg[s], sem_s[s])

        def finish_shx_issue_gather(c, s):
            @pl.when(c < nch)
            def _():
                cbase = astart + c * CH
                pltpu.make_async_copy(ids_h.at[pl.ds(cbase * 2, CH * 2)],
                                      idsb[s], sem_s[s]).wait()
                pltpu.make_async_copy(shf_h.at[pl.ds(cbase * 16, CH * 16)],
                                      shg[s], sem_s[s]).wait()
                for g in range(CH // 16):
                    sv = plsc.load_gather(idsb[s], [lane2 + (g * 32)])
                    ssv[s][pl.ds(g * 16, 16)] = sv
                pltpu.async_copy(x_h.at[ssv[s]], xg[s], sem_x[s])

        def flush(cur, accs):
            rb0 = cur * WAGG
            for k in range(28):
                plsc.addupdate(acc.at[pl.ds(rb0 + k * 16, 16)], accs[k])

        def process(c, s):
            @pl.when(c < nch)
            def _():
                cbase = astart + c * CH
                pltpu.make_async_copy(x_h.at[ssv[s]], xg[s], sem_x[s]).wait()

                def group(g, carry2):
                    cur = carry2[0]
                    accs = carry2[1:]
                    gb = g * 16
                    rv16 = plsc.load_gather(idsb[s], [lane2 + (gb * 2 + 1)])
                    for es in range(16):
                        e = gb + es
                        ge = cbase + e
                        valid = (ge >= e0) & (ge < e1)
                        row = rv16[es] - nbase
                        row = jnp.clip(row, 0, BN - 1)

                        def do_flush(cur=cur, accs=accs):
                            flush(cur, accs)
                            return (zero16,) * 28

                        def keep(accs=accs):
                            return accs

                        accs = lax.cond(row != cur, do_flush, keep)
                        cur = row
                        bvf = jnp.broadcast_to(
                            jnp.where(valid, jnp.float32(1.0),
                                      jnp.float32(0.0)), (16,))
                        shv = shg[s][pl.ds(e * 16, 16)] * bvf
                        x16 = xg[s][e, pl.ds(0, 16)]
                        new = []
                        for k in range(12):
                            new.append(accs[k]
                                       + xg[s][e, pl.ds(k * 16, 16)] * bvf)
                        for i in range(16):
                            new.append(accs[12 + i]
                                       + jnp.broadcast_to(x16[i], (16,))
                                       * shv)
                        accs = tuple(new)
                    return (cur, *accs)

                init = (jnp.int32(0),) + (zero16,) * 28
                fin = lax.fori_loop(0, CH // 16, group, init)
                flush(fin[0], fin[1:])

        issue_shx(0, 0)
        finish_shx_issue_gather(0, 0)
        issue_shx(1, 1)

        def pair(p, carry):
            c0 = 2 * p
            c1 = c0 + 1
            finish_shx_issue_gather(c1, 1)
            process(c0, 0)
            issue_shx(c0 + 2, 0)
            process(c1, 1)
            finish_shx_issue_gather(c0 + 2, 0)
            issue_shx(c1 + 2, 1)
            return carry
        lax.fori_loop(0, (nch + 1) // 2, pair, 0)
        pltpu.sync_copy(acc, agg_h.at[pl.ds(blk * BN * WAGG, BN * WAGG)])
        return carry0

    lax.fori_loop(0, 2, block, 0)


def _t1_body(agg_ref, wp_ref, wo_ref, wsc_ref, out_ref):
    h = jax.nn.gelu(jnp.dot(agg_ref[...] / 1.5, wp_ref[...],
                            preferred_element_type=jnp.float32))
    h = jnp.dot(h, wo_ref[...], preferred_element_type=jnp.float32)
    h = h + wsc_ref[...]
    out_ref[...] = jnp.concatenate(
        [h, jnp.zeros((h.shape[0], 256 - HID), jnp.float32)], axis=1)


def _t2_body(agg_ref, x_ref, wp_ref, wo_ref, wsc_ref, out_ref):
    h = jax.nn.gelu(jnp.dot(agg_ref[...] / 1.5, wp_ref[...],
                            preferred_element_type=jnp.float32))
    h = jnp.dot(h, wo_ref[...], preferred_element_type=jnp.float32)
    h = h + jnp.dot(x_ref[...], wsc_ref[...],
                    preferred_element_type=jnp.float32)
    out_ref[...] = jnp.concatenate(
        [h, jnp.zeros((h.shape[0], 256 - HID), jnp.float32)], axis=1)


def _t3_body(agg_ref, x_ref, gi_ref, wp_ref, wo_ref, wsc_ref, out_ref, pred):
    g = pl.program_id(0)

    @pl.when(g == 0)
    def _():
        pred[...] = jnp.zeros_like(pred)

    h = jax.nn.gelu(jnp.dot(agg_ref[...] / 1.5, wp_ref[...],
                            preferred_element_type=jnp.float32))
    h = jnp.dot(h, wo_ref[...], preferred_element_type=jnp.float32)
    x3 = h + jnp.dot(x_ref[...], wsc_ref[...],
                     preferred_element_type=jnp.float32)
    gi = gi_ref[0, 0, :]
    oh = (lax.broadcasted_iota(jnp.int32, (NG, 1000), 0)
          == gi[None, :]).astype(jnp.float32)
    pred[...] += jnp.dot(oh, x3, preferred_element_type=jnp.float32)

    @pl.when(g == 9)
    def _():
        p = pred[...]
        oe = p[:, 0:1] * p[:, 1:2]
        out_ref[...] = jnp.concatenate([oe, -oe, p[:, 2:8]], axis=1)


def _full(i, j):
    return pl.BlockSpec(j, lambda g: tuple(0 for _ in j)) if i is None else None


def _reshuffle(Wp):
    a = Wp[:HID]
    b = Wp[HID:].reshape(16, 15, -1)
    b = jnp.pad(b, ((0, 0), (0, 1), (0, 0)))
    return jnp.concatenate([a, b.reshape(256, -1)], axis=0)


def kernel(positions, senders, receivers, graph_ids,
           W_pre1, W_post1, W_sc1, W_pre2, W_post2, W_sc2,
           W_pre3, W_post3, W_sc3):
    senders = senders.astype(jnp.int32)
    receivers = receivers.astype(jnp.int32)
    graph_ids = graph_ids.astype(jnp.int32)
    order = jnp.argsort(receivers)
    ss = senders[order]
    sr = receivers[order]
    ss_p = jnp.pad(ss, (0, EPAD - E))
    sr_p = jnp.pad(sr, (0, EPAD - E))
    ids_p = jnp.stack([ss_p, sr_p], axis=1).reshape(-1)
    bo = jnp.searchsorted(sr, jnp.arange(NBLK + 1, dtype=jnp.int32) * BN)
    bo = bo.astype(jnp.int32)
    bo2 = jnp.stack([bo[0:64:2], bo[1:64:2], bo[2:65:2]], axis=1)
    bo2 = jnp.pad(bo2, ((0, 0), (0, 13)))
    px = positions[:, 0]
    py = positions[:, 1]
    pz = positions[:, 2]

    shf = _sh_kernel(px, py, pz, ss_p, sr_p)
    agg1 = _agg1_kernel(shf, sr_p, bo2).reshape(NPAD, 16)

    Wp1p = jnp.concatenate([W_pre1[1:16], W_pre1[0:1]], axis=0)
    x1 = pl.pallas_call(
        _t1_body, grid=(10,),
        in_specs=[pl.BlockSpec((1000, 16), lambda g: (g, 0)),
                  pl.BlockSpec((16, HID), lambda g: (0, 0)),
                  pl.BlockSpec((HID, HID), lambda g: (0, 0)),
                  pl.BlockSpec((1, HID), lambda g: (0, 0))],
        out_specs=pl.BlockSpec((1000, 256), lambda g: (g, 0)),
        out_shape=jax.ShapeDtypeStruct((N, 256), jnp.float32),
    )(agg1, Wp1p, W_post1, W_sc1)

    agg2 = _agg_kernel(x1, shf, ids_p, bo2).reshape(NPAD, WAGG)
    x2 = pl.pallas_call(
        _t2_body, grid=(10,),
        in_specs=[pl.BlockSpec((1000, WAGG), lambda g: (g, 0)),
                  pl.BlockSpec((1000, 256), lambda g: (g, 0)),
                  pl.BlockSpec((WAGG, HID), lambda g: (0, 0)),
                  pl.BlockSpec((HID, HID), lambda g: (0, 0)),
                  pl.BlockSpec((256, HID), lambda g: (0, 0))],
        out_specs=pl.BlockSpec((1000, 256), lambda g: (g, 0)),
        out_shape=jax.ShapeDtypeStruct((N, 256), jnp.float32),
    )(agg2, x1, _reshuffle(W_pre2), W_post2,
      jnp.pad(W_sc2, ((0, 256 - HID), (0, 0))))

    agg3 = _agg_kernel(x2, shf, ids_p, bo2).reshape(NPAD, WAGG)
    gi3 = graph_ids.reshape(10, 1, 1000)
    logits = pl.pallas_call(
        _t3_body, grid=(10,),
        in_specs=[pl.BlockSpec((1000, WAGG), lambda g: (g, 0)),
                  pl.BlockSpec((1000, 256), lambda g: (g, 0)),
                  pl.BlockSpec((1, 1, 1000), lambda g: (g, 0, 0)),
                  pl.BlockSpec((WAGG, 8), lambda g: (0, 0)),
                  pl.BlockSpec((8, 8), lambda g: (0, 0)),
                  pl.BlockSpec((256, 8), lambda g: (0, 0))],
        out_specs=pl.BlockSpec((NG, 8), lambda g: (0, 0)),
        out_shape=jax.ShapeDtypeStruct((NG, 8), jnp.float32),
        scratch_shapes=[pltpu.VMEM((NG, 8), jnp.float32)],
    )(agg3, x2, gi3, _reshuffle(W_pre3), W_post3,
      jnp.pad(W_sc3, ((0, 256 - HID), (0, 0))))
    return logits


# speedup vs baseline: 6.9340x; 1.0305x over previous
import functools
import numpy as np
import jax
import jax.numpy as jnp
from jax import lax
from jax.experimental import pallas as pl
from jax.experimental.pallas import tpu as pltpu
from jax.experimental.pallas import tpu_sc as plsc

N = 10000
E = 160000
NG = 64
HID = 192
BN = 157
NBLK = 64
NPAD = BN * NBLK
TPE = 5024
EPAD = 32 * TPE
CH = 64
WAGG = 448

_mesh = plsc.VectorSubcoreMesh(core_axis_name="c", subcore_axis_name="s")


def _wid():
    return lax.axis_index("s") * 2 + lax.axis_index("c")


def _rsqrt(r2):
    i = plsc.bitcast(r2, jnp.int32)
    i = jnp.int32(0x5F3759DF) - lax.shift_right_logical(i, 1)
    y = plsc.bitcast(i, jnp.float32)
    for _ in range(3):
        y = y * (1.5 - 0.5 * r2 * y * y)
    return y


@functools.partial(
    pl.kernel, mesh=_mesh,
    compiler_params=pltpu.CompilerParams(needs_layout_passes=False),
    out_type=jax.ShapeDtypeStruct((EPAD * 16,), jnp.float32),
    scratch_types=[
        pltpu.VMEM((N,), jnp.float32),
        pltpu.VMEM((N,), jnp.float32),
        pltpu.VMEM((N,), jnp.float32),
        pltpu.VMEM((TPE,), jnp.int32),
        pltpu.VMEM((TPE,), jnp.int32),
        pltpu.VMEM((TPE * 16,), jnp.float32),
    ],
)
def _sh_kernel(px_h, py_h, pz_h, ss_h, sr_h, sh_h, px, py, pz, ssv, srv, shb):
    wid = _wid()
    base = wid * TPE
    pltpu.sync_copy(px_h, px)
    pltpu.sync_copy(py_h, py)
    pltpu.sync_copy(pz_h, pz)
    pltpu.sync_copy(ss_h.at[pl.ds(base, TPE)], ssv)
    pltpu.sync_copy(sr_h.at[pl.ds(base, TPE)], srv)
    lane16 = lax.iota(jnp.int32, 16) * 16
    c3 = np.float32(np.sqrt(3.0))
    c5 = np.float32(np.sqrt(5.0))
    c7h = np.float32(np.sqrt(7.0) * 0.5)
    ones = jnp.ones((16,), jnp.float32)

    def chunk(k, carry):
        sv = ssv[pl.ds(k * 16, 16)]
        rv = srv[pl.ds(k * 16, 16)]
        dx = plsc.load_gather(px, [rv]) - plsc.load_gather(px, [sv])
        dy = plsc.load_gather(py, [rv]) - plsc.load_gather(py, [sv])
        dz = plsc.load_gather(pz, [rv]) - plsc.load_gather(pz, [sv])
        r2 = jnp.maximum(dx * dx + dy * dy + dz * dz, jnp.float32(1e-18))
        rin = _rsqrt(r2)
        ux, uy, uz = dx * rin, dy * rin, dz * rin
        x2, y2, z2 = ux * ux, uy * uy, uz * uz
        vals = (uy * c3, uz * c3, ux * c3,
                ux * uy * c5, uy * uz * c5, (1.5 * z2 - 0.5) * c5,
                ux * uz * c5, 0.5 * (x2 - y2) * c5,
                uy * (3.0 * x2 - y2) * c7h, ux * uy * uz * c7h,
                uy * (5.0 * z2 - 1.0) * c7h, uz * (5.0 * z2 - 3.0) * c7h,
                ux * (5.0 * z2 - 1.0) * c7h, uz * (x2 - y2) * c7h,
                ux * (x2 - 3.0 * y2) * c7h, ones)
        for j, v in enumerate(vals):
            plsc.store_scatter(shb, [lane16 + (k * 256 + j)], v)
        return carry

    lax.fori_loop(0, TPE // 16, chunk, 0)
    pltpu.sync_copy(shb, sh_h.at[pl.ds(base * 16, TPE * 16)])


@functools.partial(
    pl.kernel, mesh=_mesh,
    compiler_params=pltpu.CompilerParams(needs_layout_passes=False),
    out_type=jax.ShapeDtypeStruct((NPAD * 16,), jnp.float32),
    scratch_types=[
        pltpu.VMEM((16,), jnp.int32),
        pltpu.VMEM((BN * 16,), jnp.float32),
        pltpu.VMEM((CH,), jnp.int32),
        pltpu.VMEM((CH,), jnp.int32),
        pltpu.VMEM((CH * 16,), jnp.float32),
        pltpu.VMEM((CH * 16,), jnp.float32),
        pltpu.SemaphoreType.DMA,
        pltpu.SemaphoreType.DMA,
    ],
)
def _agg1_kernel(shf_h, sr_h, bo_h, agg_h, bov, acc,
                 srv0, srv1, shg0, shg1, sem_s0, sem_s1):
    wid = _wid()
    pltpu.sync_copy(bo_h.at[wid], bov)
    bvec = bov[...]
    zero16 = jnp.zeros((16,), jnp.float32)
    srv = (srv0, srv1)
    shg = (shg0, shg1)
    sem_s = (sem_s0, sem_s1)

    def block(bi, carry0):
        blk = wid * 2 + bi

        def zr(i, carry):
            acc[pl.ds(i * 16, 16)] = zero16
            return carry
        lax.fori_loop(0, BN, zr, 0)

        is0 = bi == 0
        e0 = jnp.where(is0, bvec[0], bvec[1])
        e1 = jnp.where(is0, bvec[1], bvec[2])
        astart = (e0 // 8) * 8
        nch = (e1 - astart + CH - 1) // CH
        nbase = blk * BN

        def issue(c, s):
            @pl.when(c < nch)
            def _():
                cbase = astart + c * CH
                pltpu.async_copy(sr_h.at[pl.ds(cbase, CH)], srv[s], sem_s[s])
                pltpu.async_copy(shf_h.at[pl.ds(cbase * 16, CH * 16)],
                                 shg[s], sem_s[s])

        def process(c, s):
            @pl.when(c < nch)
            def _():
                cbase = astart + c * CH
                pltpu.make_async_copy(sr_h.at[pl.ds(cbase, CH)],
                                      srv[s], sem_s[s]).wait()
                pltpu.make_async_copy(shf_h.at[pl.ds(cbase * 16, CH * 16)],
                                      shg[s], sem_s[s]).wait()

                def group(g, carry2):
                    cur, av = carry2
                    gb = g * 16
                    rv16 = srv[s][pl.ds(gb, 16)]
                    for es in range(16):
                        e = gb + es
                        ge = cbase + e
                        valid = (ge >= e0) & (ge < e1)
                        row = jnp.clip(rv16[es] - nbase, 0, BN - 1)

                        def do_flush(cur=cur, av=av):
                            plsc.addupdate(acc.at[pl.ds(cur * 16, 16)], av)
                            return zero16

                        def keep(av=av):
                            return av

                        av = lax.cond(row != cur, do_flush, keep)
                        cur = row
                        bvf = jnp.broadcast_to(
                            jnp.where(valid, jnp.float32(1.0),
                                      jnp.float32(0.0)), (16,))
                        av = av + shg[s][pl.ds(e * 16, 16)] * bvf
                    return (cur, av)

                fin = lax.fori_loop(0, CH // 16, group, (jnp.int32(0), zero16))
                plsc.addupdate(acc.at[pl.ds(fin[0] * 16, 16)], fin[1])

        issue(0, 0)
        issue(1, 1)

        def pair(p, carry):
            c0 = 2 * p
            process(c0, 0)
            issue(c0 + 2, 0)
            process(c0 + 1, 1)
            issue(c0 + 3, 1)
            return carry
        lax.fori_loop(0, (nch + 1) // 2, pair, 0)
        pltpu.sync_copy(acc, agg_h.at[pl.ds(blk * BN * 16, BN * 16)])
        return carry0

    lax.fori_loop(0, 2, block, 0)


@functools.partial(
    pl.kernel, mesh=_mesh,
    compiler_params=pltpu.CompilerParams(needs_layout_passes=False),
    out_type=jax.ShapeDtypeStruct((NPAD * WAGG,), jnp.float32),
    scratch_types=[
        pltpu.VMEM((16,), jnp.int32),
        pltpu.VMEM((BN * WAGG,), jnp.float32),
        pltpu.VMEM((CH,), jnp.int32),
        pltpu.VMEM((CH,), jnp.int32),
        pltpu.VMEM((CH * 2,), jnp.int32),
        pltpu.VMEM((CH * 2,), jnp.int32),
        pltpu.VMEM((CH, 256), jnp.float32),
        pltpu.VMEM((CH, 256), jnp.float32),
        pltpu.VMEM((CH * 16,), jnp.float32),
        pltpu.VMEM((CH * 16,), jnp.float32),
        pltpu.SemaphoreType.DMA,
        pltpu.SemaphoreType.DMA,
        pltpu.SemaphoreType.DMA,
        pltpu.SemaphoreType.DMA,
    ],
)
def _agg_kernel(x_h, shf_h, ids_h, bo_h, agg_h,
                bov, acc, ssv0, ssv1, idsb0, idsb1, xg0, xg1, shg0, shg1,
                sem_s0, sem_s1, sem_x0, sem_x1):
    wid = _wid()
    pltpu.sync_copy(bo_h.at[wid], bov)
    bvec = bov[...]
    zero16 = jnp.zeros((16,), jnp.float32)
    lane2 = lax.iota(jnp.int32, 16) * 2
    idsb = (idsb0, idsb1)
    ssv = (ssv0, ssv1)
    xg = (xg0, xg1)
    shg = (shg0, shg1)
    sem_s = (sem_s0, sem_s1)
    sem_x = (sem_x0, sem_x1)

    def block(bi, carry0):
        blk = wid * 2 + bi

        def zr(i, carry):
            acc[pl.ds(i * 16, 16)] = zero16
            return carry
        lax.fori_loop(0, BN * WAGG // 16, zr, 0)

        is0 = bi == 0
        e0 = jnp.where(is0, bvec[0], bvec[1])
        e1 = jnp.where(is0, bvec[1], bvec[2])
        astart = (e0 // 8) * 8
        nch = (e1 - astart + CH - 1) // CH
        nbase = blk * BN

        def issue_shx(c, s):
            @pl.when(c < nch)
            def _():
                cbase = astart + c * CH
                pltpu.async_copy(ids_h.at[pl.ds(cbase * 2, CH * 2)],
                                 idsb[s], sem_s[s])
                pltpu.async_copy(shf_h.at[pl.ds(cbase * 16, CH * 16)],
                                 shg[s], sem_s[s])

        def finish_shx_issue_gather(c, s):
            @pl.when(c < nch)
            def _():
                cbase = astart + c * CH
                pltpu.make_async_copy(ids_h.at[pl.ds(cbase * 2, CH * 2)],
                                      idsb[s], sem_s[s]).wait()
                pltpu.make_async_copy(shf_h.at[pl.ds(cbase * 16, CH * 16)],
                                      shg[s], sem_s[s]).wait()
                for g in range(CH // 16):
                    sv = plsc.load_gather(idsb[s], [lane2 + (g * 32)])
                    ssv[s][pl.ds(g * 16, 16)] = sv
                pltpu.async_copy(x_h.at[ssv[s]], xg[s], sem_x[s])

        def flush(cur, accs):
            rb0 = cur * WAGG
            for k in range(28):
                plsc.addupdate(acc.at[pl.ds(rb0 + k * 16, 16)], accs[k])

        def process(c, s):
            @pl.when(c < nch)
            def _():
                cbase = astart + c * CH
                pltpu.make_async_copy(x_h.at[ssv[s]], xg[s], sem_x[s]).wait()

                def group(g, carry2):
                    cur = carry2[0]
                    accs = carry2[1:]
                    gb = g * 16
                    rv16 = plsc.load_gather(idsb[s], [lane2 + (gb * 2 + 1)])
                    for es in range(16):
                        e = gb + es
                        ge = cbase + e
                        valid = (ge >= e0) & (ge < e1)
                        row = rv16[es] - nbase
                        row = jnp.clip(row, 0, BN - 1)

                        def do_flush(cur=cur, accs=accs):
                            flush(cur, accs)
                            return (zero16,) * 28

                        def keep(accs=accs):
                            return accs

                        accs = lax.cond(row != cur, do_flush, keep)
                        cur = row
                        bvf = jnp.broadcast_to(
                            jnp.where(valid, jnp.float32(1.0),
                                      jnp.float32(0.0)), (16,))
                        shv = shg[s][pl.ds(e * 16, 16)] * bvf
                        x16 = xg[s][e, pl.ds(0, 16)]
                        new = []
                        for k in range(12):
                            new.append(accs[k]
                                       + xg[s][e, pl.ds(k * 16, 16)] * bvf)
                        for i in range(16):
                            new.append(accs[12 + i]
                                       + jnp.broadcast_to(x16[i], (16,))
                                       * shv)
                        accs = tuple(new)
                    return (cur, *accs)

                init = (jnp.int32(0),) + (zero16,) * 28
                fin = lax.fori_loop(0, CH // 16, group, init)
                flush(fin[0], fin[1:])

        issue_shx(0, 0)
        finish_shx_issue_gather(0, 0)
        issue_shx(1, 1)

        def pair(p, carry):
            c0 = 2 * p
            c1 = c0 + 1
            finish_shx_issue_gather(c1, 1)
            process(c0, 0)
            issue_shx(c0 + 2, 0)
            process(c1, 1)
            finish_shx_issue_gather(c0 + 2, 0)
            issue_shx(c1 + 2, 1)
            return carry
        lax.fori_loop(0, (nch + 1) // 2, pair, 0)
        pltpu.sync_copy(acc, agg_h.at[pl.ds(blk * BN * WAGG, BN * WAGG)])
        return carry0

    lax.fori_loop(0, 2, block, 0)


def _t1_body(agg_ref, wp_ref, wo_ref, wsc_ref, out_ref):
    h = jax.nn.gelu(jnp.dot(agg_ref[...] / 1.5, wp_ref[...],
                            preferred_element_type=jnp.float32))
    h = jnp.dot(h, wo_ref[...], preferred_element_type=jnp.float32)
    h = h + wsc_ref[...]
    out_ref[...] = jnp.concatenate(
        [h, jnp.zeros((h.shape[0], 256 - HID), jnp.float32)], axis=1)


def _t2_body(agg_ref, x_ref, wp_ref, wo_ref, wsc_ref, out_ref):
    h = jax.nn.gelu(jnp.dot(agg_ref[...] / 1.5, wp_ref[...],
                            preferred_element_type=jnp.float32))
    h = jnp.dot(h, wo_ref[...], preferred_element_type=jnp.float32)
    h = h + jnp.dot(x_ref[...], wsc_ref[...],
                    preferred_element_type=jnp.float32)
    out_ref[...] = jnp.concatenate(
        [h, jnp.zeros((h.shape[0], 256 - HID), jnp.float32)], axis=1)


def _t3_body(agg_ref, x_ref, gi_ref, wp_ref, wo_ref, wsc_ref, out_ref, pred):
    g = pl.program_id(0)

    @pl.when(g == 0)
    def _():
        pred[...] = jnp.zeros_like(pred)

    h = jax.nn.gelu(jnp.dot(agg_ref[...] / 1.5, wp_ref[...],
                            preferred_element_type=jnp.float32))
    h = jnp.dot(h, wo_ref[...], preferred_element_type=jnp.float32)
    x3 = h + jnp.dot(x_ref[...], wsc_ref[...],
                     preferred_element_type=jnp.float32)
    gi = gi_ref[0, 0, :]
    oh = (lax.broadcasted_iota(jnp.int32, (NG, 1000), 0)
          == gi[None, :]).astype(jnp.float32)
    pred[...] += jnp.dot(oh, x3, preferred_element_type=jnp.float32)

    @pl.when(g == 9)
    def _():
        p = pred[...]
        oe = p[:, 0:1] * p[:, 1:2]
        out_ref[...] = jnp.concatenate([oe, -oe, p[:, 2:8]], axis=1)


def _full(i, j):
    return pl.BlockSpec(j, lambda g: tuple(0 for _ in j)) if i is None else None


def _reshuffle(Wp):
    a = Wp[:HID]
    b = Wp[HID:].reshape(16, 15, -1)
    b = jnp.pad(b, ((0, 0), (0, 1), (0, 0)))
    return jnp.concatenate([a, b.reshape(256, -1)], axis=0)


def kernel(positions, senders, receivers, graph_ids,
           W_pre1, W_post1, W_sc1, W_pre2, W_post2, W_sc2,
           W_pre3, W_post3, W_sc3):
    senders = senders.astype(jnp.int32)
    receivers = receivers.astype(jnp.int32)
    graph_ids = graph_ids.astype(jnp.int32)
    order = jnp.argsort(receivers)
    ss = senders[order]
    sr = receivers[order]
    ss_p = jnp.pad(ss, (0, EPAD - E))
    sr_p = jnp.pad(sr, (0, EPAD - E))
    ids_p = jnp.stack([ss_p, sr_p], axis=1).reshape(-1)
    bo = jnp.searchsorted(sr, jnp.arange(NBLK + 1, dtype=jnp.int32) * BN)
    bo = bo.astype(jnp.int32)
    bo2 = jnp.stack([bo[0:64:2], bo[1:64:2], bo[2:65:2]], axis=1)
    bo2 = jnp.pad(bo2, ((0, 0), (0, 13)))
    px = positions[:, 0]
    py = positions[:, 1]
    pz = positions[:, 2]

    shf = _sh_kernel(px, py, pz, ss_p, sr_p)
    agg1 = _agg1_kernel(shf, sr_p, bo2).reshape(NPAD, 16)

    Wp1p = jnp.concatenate([W_pre1[1:16], W_pre1[0:1]], axis=0)
    x1 = pl.pallas_call(
        _t1_body, grid=(10,),
        in_specs=[pl.BlockSpec((1000, 16), lambda g: (g, 0)),
                  pl.BlockSpec((16, HID), lambda g: (0, 0)),
                  pl.BlockSpec((HID, HID), lambda g: (0, 0)),
                  pl.BlockSpec((1, HID), lambda g: (0, 0))],
        out_specs=pl.BlockSpec((1000, 256), lambda g: (g, 0)),
        out_shape=jax.ShapeDtypeStruct((N, 256), jnp.float32),
    )(agg1, Wp1p, W_post1, W_sc1)

    agg2 = _agg_kernel(x1, shf, ids_p, bo2).reshape(NPAD, WAGG)
    x2 = pl.pallas_call(
        _t2_body, grid=(10,),
        in_specs=[pl.BlockSpec((1000, WAGG), lambda g: (g, 0)),
                  pl.BlockSpec((1000, 256), lambda g: (g, 0)),
                  pl.BlockSpec((WAGG, HID), lambda g: (0, 0)),
                  pl.BlockSpec((HID, HID), lambda g: (0, 0)),
                  pl.BlockSpec((256, HID), lambda g: (0, 0))],
        out_specs=pl.BlockSpec((1000, 256), lambda g: (g, 0)),
        out_shape=jax.ShapeDtypeStruct((N, 256), jnp.float32),
    )(agg2, x1, _reshuffle(W_pre2), W_post2,
      jnp.pad(W_sc2, ((0, 256 - HID), (0, 0))))

    agg3 = _agg_kernel(x2, shf, ids_p, bo2).reshape(NPAD, WAGG)
    gi3 = graph_ids.reshape(10, 1, 1000)
    logits = pl.pallas_call(
        _t3_body, grid=(10,),
        in_specs=[pl.BlockSpec((1000, WAGG), lambda g: (g, 0)),
                  pl.BlockSpec((1000, 256), lambda g: (g, 0)),
                  pl.BlockSpec((1, 1, 1000), lambda g: (g, 0, 0)),
                  pl.BlockSpec((WAGG, 8), lambda g: (0, 0)),
                  pl.BlockSpec((8, 8), lambda g: (0, 0)),
                  pl.BlockSpec((256, 8), lambda g: (0, 0))],
        out_specs=pl.BlockSpec((NG, 8), lambda g: (0, 0)),
        out_shape=jax.ShapeDtypeStruct((NG, 8), jnp.float32),
        scratch_shapes=[pltpu.VMEM((NG, 8), jnp.float32)],
    )(agg3, x2, gi3, _reshuffle(W_pre3), W_post3,
      jnp.pad(W_sc3, ((0, 256 - HID), (0, 0))))
    return logits


# speedup vs baseline: 7.2954x; 1.0521x over previous
import functools
import numpy as np
import jax
import jax.numpy as jnp
from jax import lax
from jax.experimental import pallas as pl
from jax.experimental.pallas import tpu as pltpu
from jax.experimental.pallas import tpu_sc as plsc

N = 10000
E = 160000
NG = 64
HID = 192
BN = 157
NBLK = 64
NPAD = BN * NBLK
TPE = 5024
EPAD = 32 * TPE
CH = 64
WAGG = 448

_mesh = plsc.VectorSubcoreMesh(core_axis_name="c", subcore_axis_name="s")


def _wid():
    return lax.axis_index("s") * 2 + lax.axis_index("c")


def _rsqrt(r2):
    i = plsc.bitcast(r2, jnp.int32)
    i = jnp.int32(0x5F3759DF) - lax.shift_right_logical(i, 1)
    y = plsc.bitcast(i, jnp.float32)
    for _ in range(3):
        y = y * (1.5 - 0.5 * r2 * y * y)
    return y


@functools.partial(
    pl.kernel, mesh=_mesh,
    compiler_params=pltpu.CompilerParams(needs_layout_passes=False),
    out_type=jax.ShapeDtypeStruct((EPAD * 16,), jnp.float32),
    scratch_types=[
        pltpu.VMEM((N,), jnp.float32),
        pltpu.VMEM((N,), jnp.float32),
        pltpu.VMEM((N,), jnp.float32),
        pltpu.VMEM((TPE,), jnp.int32),
        pltpu.VMEM((TPE,), jnp.int32),
        pltpu.VMEM((TPE * 16,), jnp.float32),
    ],
)
def _sh_kernel(px_h, py_h, pz_h, ss_h, sr_h, sh_h, px, py, pz, ssv, srv, shb):
    wid = _wid()
    base = wid * TPE
    pltpu.sync_copy(px_h, px)
    pltpu.sync_copy(py_h, py)
    pltpu.sync_copy(pz_h, pz)
    pltpu.sync_copy(ss_h.at[pl.ds(base, TPE)], ssv)
    pltpu.sync_copy(sr_h.at[pl.ds(base, TPE)], srv)
    lane16 = lax.iota(jnp.int32, 16) * 16
    c3 = np.float32(np.sqrt(3.0))
    c5 = np.float32(np.sqrt(5.0))
    c7h = np.float32(np.sqrt(7.0) * 0.5)
    ones = jnp.ones((16,), jnp.float32)

    def chunk(k, carry):
        sv = ssv[pl.ds(k * 16, 16)]
        rv = srv[pl.ds(k * 16, 16)]
        dx = plsc.load_gather(px, [rv]) - plsc.load_gather(px, [sv])
        dy = plsc.load_gather(py, [rv]) - plsc.load_gather(py, [sv])
        dz = plsc.load_gather(pz, [rv]) - plsc.load_gather(pz, [sv])
        r2 = jnp.maximum(dx * dx + dy * dy + dz * dz, jnp.float32(1e-18))
        rin = _rsqrt(r2)
        ux, uy, uz = dx * rin, dy * rin, dz * rin
        x2, y2, z2 = ux * ux, uy * uy, uz * uz
        vals = (uy * c3, uz * c3, ux * c3,
                ux * uy * c5, uy * uz * c5, (1.5 * z2 - 0.5) * c5,
                ux * uz * c5, 0.5 * (x2 - y2) * c5,
                uy * (3.0 * x2 - y2) * c7h, ux * uy * uz * c7h,
                uy * (5.0 * z2 - 1.0) * c7h, uz * (5.0 * z2 - 3.0) * c7h,
                ux * (5.0 * z2 - 1.0) * c7h, uz * (x2 - y2) * c7h,
                ux * (x2 - 3.0 * y2) * c7h, ones)
        for j, v in enumerate(vals):
            plsc.store_scatter(shb, [lane16 + (k * 256 + j)], v)
        return carry

    lax.fori_loop(0, TPE // 16, chunk, 0)
    pltpu.sync_copy(shb, sh_h.at[pl.ds(base * 16, TPE * 16)])


@functools.partial(
    pl.kernel, mesh=_mesh,
    compiler_params=pltpu.CompilerParams(needs_layout_passes=False),
    out_type=jax.ShapeDtypeStruct((NPAD * 16,), jnp.float32),
    scratch_types=[
        pltpu.VMEM((16,), jnp.int32),
        pltpu.VMEM((BN * 16,), jnp.float32),
        pltpu.VMEM((CH,), jnp.int32),
        pltpu.VMEM((CH,), jnp.int32),
        pltpu.VMEM((CH * 16,), jnp.float32),
        pltpu.VMEM((CH * 16,), jnp.float32),
        pltpu.SemaphoreType.DMA,
        pltpu.SemaphoreType.DMA,
    ],
)
def _agg1_kernel(shf_h, sr_h, bo_h, agg_h, bov, acc,
                 srv0, srv1, shg0, shg1, sem_s0, sem_s1):
    wid = _wid()
    pltpu.sync_copy(bo_h.at[wid], bov)
    bvec = bov[...]
    zero16 = jnp.zeros((16,), jnp.float32)
    srv = (srv0, srv1)
    shg = (shg0, shg1)
    sem_s = (sem_s0, sem_s1)

    def block(bi, carry0):
        blk = wid * 2 + bi

        def zr(i, carry):
            acc[pl.ds(i * 16, 16)] = zero16
            return carry
        lax.fori_loop(0, BN, zr, 0)

        is0 = bi == 0
        e0 = jnp.where(is0, bvec[0], bvec[1])
        e1 = jnp.where(is0, bvec[1], bvec[2])
        astart = (e0 // 8) * 8
        nch = (e1 - astart + CH - 1) // CH
        nbase = blk * BN

        def issue(c, s):
            @pl.when(c < nch)
            def _():
                cbase = astart + c * CH
                pltpu.async_copy(sr_h.at[pl.ds(cbase, CH)], srv[s], sem_s[s])
                pltpu.async_copy(shf_h.at[pl.ds(cbase * 16, CH * 16)],
                                 shg[s], sem_s[s])

        def process(c, s):
            @pl.when(c < nch)
            def _():
                cbase = astart + c * CH
                pltpu.make_async_copy(sr_h.at[pl.ds(cbase, CH)],
                                      srv[s], sem_s[s]).wait()
                pltpu.make_async_copy(shf_h.at[pl.ds(cbase * 16, CH * 16)],
                                      shg[s], sem_s[s]).wait()

                def group(g, carry2):
                    cur, av = carry2
                    gb = g * 16
                    rv16 = srv[s][pl.ds(gb, 16)]
                    for es in range(16):
                        e = gb + es
                        ge = cbase + e
                        valid = (ge >= e0) & (ge < e1)
                        row = jnp.clip(rv16[es] - nbase, 0, BN - 1)

                        def do_flush(cur=cur, av=av):
                            plsc.addupdate(acc.at[pl.ds(cur * 16, 16)], av)
                            return zero16

                        def keep(av=av):
                            return av

                        av = lax.cond(row != cur, do_flush, keep)
                        cur = row
                        bvf = jnp.broadcast_to(
                            jnp.where(valid, jnp.float32(1.0),
                                      jnp.float32(0.0)), (16,))
                        av = av + shg[s][pl.ds(e * 16, 16)] * bvf
                    return (cur, av)

                fin = lax.fori_loop(0, CH // 16, group, (jnp.int32(0), zero16))
                plsc.addupdate(acc.at[pl.ds(fin[0] * 16, 16)], fin[1])

        issue(0, 0)
        issue(1, 1)

        def pair(p, carry):
            c0 = 2 * p
            process(c0, 0)
            issue(c0 + 2, 0)
            process(c0 + 1, 1)
            issue(c0 + 3, 1)
            return carry
        lax.fori_loop(0, (nch + 1) // 2, pair, 0)
        pltpu.sync_copy(acc, agg_h.at[pl.ds(blk * BN * 16, BN * 16)])
        return carry0

    lax.fori_loop(0, 2, block, 0)


@functools.partial(
    pl.kernel, mesh=_mesh,
    compiler_params=pltpu.CompilerParams(needs_layout_passes=False),
    out_type=jax.ShapeDtypeStruct((NPAD * WAGG,), jnp.float32),
    scratch_types=[
        pltpu.VMEM((16,), jnp.int32),
        pltpu.VMEM((BN * WAGG,), jnp.float32),
        pltpu.VMEM((CH,), jnp.int32),
        pltpu.VMEM((CH,), jnp.int32),
        pltpu.VMEM((CH,), jnp.int32),
        pltpu.VMEM((CH,), jnp.int32),
        pltpu.VMEM((CH, 256), jnp.float32),
        pltpu.VMEM((CH, 256), jnp.float32),
        pltpu.VMEM((CH * 16,), jnp.float32),
        pltpu.VMEM((CH * 16,), jnp.float32),
        pltpu.SemaphoreType.DMA,
        pltpu.SemaphoreType.DMA,
        pltpu.SemaphoreType.DMA,
        pltpu.SemaphoreType.DMA,
    ],
)
def _agg_kernel(x_h, shf_h, ss_h, sr_h, bo_h, agg_h,
                bov, acc, ssv0, ssv1, srvb0, srvb1, xg0, xg1, shg0, shg1,
                sem_s0, sem_s1, sem_x0, sem_x1):
    wid = _wid()
    pltpu.sync_copy(bo_h.at[wid], bov)
    bvec = bov[...]
    zero16 = jnp.zeros((16,), jnp.float32)
    srvb = (srvb0, srvb1)
    ssv = (ssv0, ssv1)
    xg = (xg0, xg1)
    shg = (shg0, shg1)
    sem_s = (sem_s0, sem_s1)
    sem_x = (sem_x0, sem_x1)

    def block(bi, carry0):
        blk = wid * 2 + bi

        def zr(i, carry):
            acc[pl.ds(i * 16, 16)] = zero16
            return carry
        lax.fori_loop(0, BN * WAGG // 16, zr, 0)

        is0 = bi == 0
        e0 = jnp.where(is0, bvec[0], bvec[1])
        e1 = jnp.where(is0, bvec[1], bvec[2])
        astart = (e0 // 8) * 8
        nch = (e1 - astart + CH - 1) // CH
        nbase = blk * BN

        def issue_shx(c, s):
            @pl.when(c < nch)
            def _():
                cbase = astart + c * CH
                pltpu.async_copy(ss_h.at[pl.ds(cbase, CH)], ssv[s], sem_s[s])
                pltpu.async_copy(sr_h.at[pl.ds(cbase, CH)], srvb[s], sem_s[s])
                pltpu.async_copy(shf_h.at[pl.ds(cbase * 16, CH * 16)],
                                 shg[s], sem_s[s])

        def finish_shx_issue_gather(c, s):
            @pl.when(c < nch)
            def _():
                cbase = astart + c * CH
                pltpu.make_async_copy(ss_h.at[pl.ds(cbase, CH)],
                                      ssv[s], sem_s[s]).wait()
                pltpu.make_async_copy(sr_h.at[pl.ds(cbase, CH)],
                                      srvb[s], sem_s[s]).wait()
                pltpu.make_async_copy(shf_h.at[pl.ds(cbase * 16, CH * 16)],
                                      shg[s], sem_s[s]).wait()
                pltpu.async_copy(x_h.at[ssv[s]], xg[s], sem_x[s])

        def flush(cur, accs):
            rb0 = cur * WAGG
            for k in range(28):
                plsc.addupdate(acc.at[pl.ds(rb0 + k * 16, 16)], accs[k])

        def process(c, s):
            @pl.when(c < nch)
            def _():
                cbase = astart + c * CH
                pltpu.make_async_copy(x_h.at[ssv[s]], xg[s], sem_x[s]).wait()

                def group(g, carry2):
                    cur = carry2[0]
                    accs = carry2[1:]
                    gb = g * 16
                    rv16 = srvb[s][pl.ds(gb, 16)]
                    for es in range(16):
                        e = gb + es
                        ge = cbase + e
                        valid = (ge >= e0) & (ge < e1)
                        row = rv16[es] - nbase
                        row = jnp.clip(row, 0, BN - 1)

                        def do_flush(cur=cur, accs=accs):
                            flush(cur, accs)
                            return (zero16,) * 28

                        def keep(accs=accs):
                            return accs

                        accs = lax.cond(row != cur, do_flush, keep)
                        cur = row
                        bvf = jnp.broadcast_to(
                            jnp.where(valid, jnp.float32(1.0),
                                      jnp.float32(0.0)), (16,))
                        shv = shg[s][pl.ds(e * 16, 16)] * bvf
                        x16 = xg[s][e, pl.ds(0, 16)]
                        new = []
                        for k in range(12):
                            new.append(accs[k]
                                       + xg[s][e, pl.ds(k * 16, 16)] * bvf)
                        for i in range(16):
                            new.append(accs[12 + i]
                                       + jnp.broadcast_to(x16[i], (16,))
                                       * shv)
                        accs = tuple(new)
                    return (cur, *accs)

                init = (jnp.int32(0),) + (zero16,) * 28
                fin = lax.fori_loop(0, CH // 16, group, init)
                flush(fin[0], fin[1:])

        issue_shx(0, 0)
        finish_shx_issue_gather(0, 0)
        issue_shx(1, 1)

        def pair(p, carry):
            c0 = 2 * p
            c1 = c0 + 1
            finish_shx_issue_gather(c1, 1)
            process(c0, 0)
            issue_shx(c0 + 2, 0)
            process(c1, 1)
            finish_shx_issue_gather(c0 + 2, 0)
            issue_shx(c1 + 2, 1)
            return carry
        lax.fori_loop(0, (nch + 1) // 2, pair, 0)
        pltpu.sync_copy(acc, agg_h.at[pl.ds(blk * BN * WAGG, BN * WAGG)])
        return carry0

    lax.fori_loop(0, 2, block, 0)


def _t1_body(agg_ref, wp_ref, wo_ref, wsc_ref, out_ref):
    h = jax.nn.gelu(jnp.dot(agg_ref[...] / 1.5, wp_ref[...],
                            preferred_element_type=jnp.float32))
    h = jnp.dot(h, wo_ref[...], preferred_element_type=jnp.float32)
    h = h + wsc_ref[...]
    out_ref[...] = jnp.concatenate(
        [h, jnp.zeros((h.shape[0], 256 - HID), jnp.float32)], axis=1)


def _t2_body(agg_ref, x_ref, wp_ref, wo_ref, wsc_ref, out_ref):
    h = jax.nn.gelu(jnp.dot(agg_ref[...] / 1.5, wp_ref[...],
                            preferred_element_type=jnp.float32))
    h = jnp.dot(h, wo_ref[...], preferred_element_type=jnp.float32)
    h = h + jnp.dot(x_ref[...], wsc_ref[...],
                    preferred_element_type=jnp.float32)
    out_ref[...] = jnp.concatenate(
        [h, jnp.zeros((h.shape[0], 256 - HID), jnp.float32)], axis=1)


def _t3_body(agg_ref, x_ref, gi_ref, wp_ref, wo_ref, wsc_ref, out_ref, pred):
    g = pl.program_id(0)

    @pl.when(g == 0)
    def _():
        pred[...] = jnp.zeros_like(pred)

    h = jax.nn.gelu(jnp.dot(agg_ref[...] / 1.5, wp_ref[...],
                            preferred_element_type=jnp.float32))
    h = jnp.dot(h, wo_ref[...], preferred_element_type=jnp.float32)
    x3 = h + jnp.dot(x_ref[...], wsc_ref[...],
                     preferred_element_type=jnp.float32)
    gi = gi_ref[0, 0, :]
    oh = (lax.broadcasted_iota(jnp.int32, (NG, 1000), 0)
          == gi[None, :]).astype(jnp.float32)
    pred[...] += jnp.dot(oh, x3, preferred_element_type=jnp.float32)

    @pl.when(g == 9)
    def _():
        p = pred[...]
        oe = p[:, 0:1] * p[:, 1:2]
        out_ref[...] = jnp.concatenate([oe, -oe, p[:, 2:8]], axis=1)


def _full(i, j):
    return pl.BlockSpec(j, lambda g: tuple(0 for _ in j)) if i is None else None


def _reshuffle(Wp):
    a = Wp[:HID]
    b = Wp[HID:].reshape(16, 15, -1)
    b = jnp.pad(b, ((0, 0), (0, 1), (0, 0)))
    return jnp.concatenate([a, b.reshape(256, -1)], axis=0)


def kernel(positions, senders, receivers, graph_ids,
           W_pre1, W_post1, W_sc1, W_pre2, W_post2, W_sc2,
           W_pre3, W_post3, W_sc3):
    senders = senders.astype(jnp.int32)
    receivers = receivers.astype(jnp.int32)
    graph_ids = graph_ids.astype(jnp.int32)
    order = jnp.argsort(receivers)
    ss = senders[order]
    sr = receivers[order]
    ss_p = jnp.pad(ss, (0, EPAD - E))
    sr_p = jnp.pad(sr, (0, EPAD - E))
    bo = jnp.searchsorted(sr, jnp.arange(NBLK + 1, dtype=jnp.int32) * BN)
    bo = bo.astype(jnp.int32)
    bo2 = jnp.stack([bo[0:64:2], bo[1:64:2], bo[2:65:2]], axis=1)
    bo2 = jnp.pad(bo2, ((0, 0), (0, 13)))
    px = positions[:, 0]
    py = positions[:, 1]
    pz = positions[:, 2]

    shf = _sh_kernel(px, py, pz, ss_p, sr_p)
    agg1 = _agg1_kernel(shf, sr_p, bo2).reshape(NPAD, 16)

    Wp1p = jnp.concatenate([W_pre1[1:16], W_pre1[0:1]], axis=0)
    x1 = pl.pallas_call(
        _t1_body, grid=(10,),
        in_specs=[pl.BlockSpec((1000, 16), lambda g: (g, 0)),
                  pl.BlockSpec((16, HID), lambda g: (0, 0)),
                  pl.BlockSpec((HID, HID), lambda g: (0, 0)),
                  pl.BlockSpec((1, HID), lambda g: (0, 0))],
        out_specs=pl.BlockSpec((1000, 256), lambda g: (g, 0)),
        out_shape=jax.ShapeDtypeStruct((N, 256), jnp.float32),
    )(agg1, Wp1p, W_post1, W_sc1)

    agg2 = _agg_kernel(x1, shf, ss_p, sr_p, bo2).reshape(NPAD, WAGG)
    x2 = pl.pallas_call(
        _t2_body, grid=(10,),
        in_specs=[pl.BlockSpec((1000, WAGG), lambda g: (g, 0)),
                  pl.BlockSpec((1000, 256), lambda g: (g, 0)),
                  pl.BlockSpec((WAGG, HID), lambda g: (0, 0)),
                  pl.BlockSpec((HID, HID), lambda g: (0, 0)),
                  pl.BlockSpec((256, HID), lambda g: (0, 0))],
        out_specs=pl.BlockSpec((1000, 256), lambda g: (g, 0)),
        out_shape=jax.ShapeDtypeStruct((N, 256), jnp.float32),
    )(agg2, x1, _reshuffle(W_pre2), W_post2,
      jnp.pad(W_sc2, ((0, 256 - HID), (0, 0))))

    agg3 = _agg_kernel(x2, shf, ss_p, sr_p, bo2).reshape(NPAD, WAGG)
    gi3 = graph_ids.reshape(10, 1, 1000)
    logits = pl.pallas_call(
        _t3_body, grid=(10,),
        in_specs=[pl.BlockSpec((1000, WAGG), lambda g: (g, 0)),
                  pl.BlockSpec((1000, 256), lambda g: (g, 0)),
                  pl.BlockSpec((1, 1, 1000), lambda g: (g, 0, 0)),
                  pl.BlockSpec((WAGG, 8), lambda g: (0, 0)),
                  pl.BlockSpec((8, 8), lambda g: (0, 0)),
                  pl.BlockSpec((256, 8), lambda g: (0, 0))],
        out_specs=pl.BlockSpec((NG, 8), lambda g: (0, 0)),
        out_shape=jax.ShapeDtypeStruct((NG, 8), jnp.float32),
        scratch_shapes=[pltpu.VMEM((NG, 8), jnp.float32)],
    )(agg3, x2, gi3, _reshuffle(W_pre3), W_post3,
      jnp.pad(W_sc3, ((0, 256 - HID), (0, 0))))
    return logits


# speedup vs baseline: 7.6429x; 1.0476x over previous
import functools
import numpy as np
import jax
import jax.numpy as jnp
from jax import lax
from jax.experimental import pallas as pl
from jax.experimental.pallas import tpu as pltpu
from jax.experimental.pallas import tpu_sc as plsc

N = 10000
E = 160000
NG = 64
HID = 192
BN = 157
NBLK = 64
NPAD = BN * NBLK
TPE = 5024
EPAD = 32 * TPE
CH = 96
WAGG = 448

_mesh = plsc.VectorSubcoreMesh(core_axis_name="c", subcore_axis_name="s")


def _wid():
    return lax.axis_index("s") * 2 + lax.axis_index("c")


def _rsqrt(r2):
    i = plsc.bitcast(r2, jnp.int32)
    i = jnp.int32(0x5F3759DF) - lax.shift_right_logical(i, 1)
    y = plsc.bitcast(i, jnp.float32)
    for _ in range(3):
        y = y * (1.5 - 0.5 * r2 * y * y)
    return y


@functools.partial(
    pl.kernel, mesh=_mesh,
    compiler_params=pltpu.CompilerParams(needs_layout_passes=False),
    out_type=jax.ShapeDtypeStruct((EPAD * 16,), jnp.float32),
    scratch_types=[
        pltpu.VMEM((N,), jnp.float32),
        pltpu.VMEM((N,), jnp.float32),
        pltpu.VMEM((N,), jnp.float32),
        pltpu.VMEM((TPE,), jnp.int32),
        pltpu.VMEM((TPE,), jnp.int32),
        pltpu.VMEM((TPE * 16,), jnp.float32),
    ],
)
def _sh_kernel(px_h, py_h, pz_h, ss_h, sr_h, sh_h, px, py, pz, ssv, srv, shb):
    wid = _wid()
    base = wid * TPE
    pltpu.sync_copy(px_h, px)
    pltpu.sync_copy(py_h, py)
    pltpu.sync_copy(pz_h, pz)
    pltpu.sync_copy(ss_h.at[pl.ds(base, TPE)], ssv)
    pltpu.sync_copy(sr_h.at[pl.ds(base, TPE)], srv)
    lane16 = lax.iota(jnp.int32, 16) * 16
    c3 = np.float32(np.sqrt(3.0))
    c5 = np.float32(np.sqrt(5.0))
    c7h = np.float32(np.sqrt(7.0) * 0.5)
    ones = jnp.ones((16,), jnp.float32)

    def chunk(k, carry):
        sv = ssv[pl.ds(k * 16, 16)]
        rv = srv[pl.ds(k * 16, 16)]
        dx = plsc.load_gather(px, [rv]) - plsc.load_gather(px, [sv])
        dy = plsc.load_gather(py, [rv]) - plsc.load_gather(py, [sv])
        dz = plsc.load_gather(pz, [rv]) - plsc.load_gather(pz, [sv])
        r2 = jnp.maximum(dx * dx + dy * dy + dz * dz, jnp.float32(1e-18))
        rin = _rsqrt(r2)
        ux, uy, uz = dx * rin, dy * rin, dz * rin
        x2, y2, z2 = ux * ux, uy * uy, uz * uz
        vals = (uy * c3, uz * c3, ux * c3,
                ux * uy * c5, uy * uz * c5, (1.5 * z2 - 0.5) * c5,
                ux * uz * c5, 0.5 * (x2 - y2) * c5,
                uy * (3.0 * x2 - y2) * c7h, ux * uy * uz * c7h,
                uy * (5.0 * z2 - 1.0) * c7h, uz * (5.0 * z2 - 3.0) * c7h,
                ux * (5.0 * z2 - 1.0) * c7h, uz * (x2 - y2) * c7h,
                ux * (x2 - 3.0 * y2) * c7h, ones)
        for j, v in enumerate(vals):
            plsc.store_scatter(shb, [lane16 + (k * 256 + j)], v)
        return carry

    lax.fori_loop(0, TPE // 16, chunk, 0)
    pltpu.sync_copy(shb, sh_h.at[pl.ds(base * 16, TPE * 16)])


@functools.partial(
    pl.kernel, mesh=_mesh,
    compiler_params=pltpu.CompilerParams(needs_layout_passes=False),
    out_type=jax.ShapeDtypeStruct((NPAD * 16,), jnp.float32),
    scratch_types=[
        pltpu.VMEM((16,), jnp.int32),
        pltpu.VMEM((BN * 16,), jnp.float32),
        pltpu.VMEM((CH,), jnp.int32),
        pltpu.VMEM((CH,), jnp.int32),
        pltpu.VMEM((CH * 16,), jnp.float32),
        pltpu.VMEM((CH * 16,), jnp.float32),
        pltpu.SemaphoreType.DMA,
        pltpu.SemaphoreType.DMA,
    ],
)
def _agg1_kernel(shf_h, sr_h, bo_h, agg_h, bov, acc,
                 srv0, srv1, shg0, shg1, sem_s0, sem_s1):
    wid = _wid()
    pltpu.sync_copy(bo_h.at[wid], bov)
    bvec = bov[...]
    zero16 = jnp.zeros((16,), jnp.float32)
    srv = (srv0, srv1)
    shg = (shg0, shg1)
    sem_s = (sem_s0, sem_s1)

    def block(bi, carry0):
        blk = wid * 2 + bi

        def zr(i, carry):
            acc[pl.ds(i * 16, 16)] = zero16
            return carry
        lax.fori_loop(0, BN, zr, 0)

        is0 = bi == 0
        e0 = jnp.where(is0, bvec[0], bvec[1])
        e1 = jnp.where(is0, bvec[1], bvec[2])
        astart = (e0 // 8) * 8
        nch = (e1 - astart + CH - 1) // CH
        nbase = blk * BN

        def issue(c, s):
            @pl.when(c < nch)
            def _():
                cbase = astart + c * CH
                pltpu.async_copy(sr_h.at[pl.ds(cbase, CH)], srv[s], sem_s[s])
                pltpu.async_copy(shf_h.at[pl.ds(cbase * 16, CH * 16)],
                                 shg[s], sem_s[s])

        def process(c, s):
            @pl.when(c < nch)
            def _():
                cbase = astart + c * CH
                pltpu.make_async_copy(sr_h.at[pl.ds(cbase, CH)],
                                      srv[s], sem_s[s]).wait()
                pltpu.make_async_copy(shf_h.at[pl.ds(cbase * 16, CH * 16)],
                                      shg[s], sem_s[s]).wait()

                def group(g, carry2):
                    cur, av = carry2
                    gb = g * 16
                    rv16 = srv[s][pl.ds(gb, 16)]
                    for es in range(16):
                        e = gb + es
                        ge = cbase + e
                        valid = (ge >= e0) & (ge < e1)
                        row = jnp.clip(rv16[es] - nbase, 0, BN - 1)

                        def do_flush(cur=cur, av=av):
                            plsc.addupdate(acc.at[pl.ds(cur * 16, 16)], av)
                            return zero16

                        def keep(av=av):
                            return av

                        av = lax.cond(row != cur, do_flush, keep)
                        cur = row
                        bvf = jnp.broadcast_to(
                            jnp.where(valid, jnp.float32(1.0),
                                      jnp.float32(0.0)), (16,))
                        av = av + shg[s][pl.ds(e * 16, 16)] * bvf
                    return (cur, av)

                fin = lax.fori_loop(0, CH // 16, group, (jnp.int32(0), zero16))
                plsc.addupdate(acc.at[pl.ds(fin[0] * 16, 16)], fin[1])

        issue(0, 0)
        issue(1, 1)

        def pair(p, carry):
            c0 = 2 * p
            process(c0, 0)
            issue(c0 + 2, 0)
            process(c0 + 1, 1)
            issue(c0 + 3, 1)
            return carry
        lax.fori_loop(0, (nch + 1) // 2, pair, 0)
        pltpu.sync_copy(acc, agg_h.at[pl.ds(blk * BN * 16, BN * 16)])
        return carry0

    lax.fori_loop(0, 2, block, 0)


@functools.partial(
    pl.kernel, mesh=_mesh,
    compiler_params=pltpu.CompilerParams(needs_layout_passes=False),
    out_type=jax.ShapeDtypeStruct((NPAD * WAGG,), jnp.float32),
    scratch_types=[
        pltpu.VMEM((16,), jnp.int32),
        pltpu.VMEM((BN * WAGG,), jnp.float32),
        pltpu.VMEM((CH,), jnp.int32),
        pltpu.VMEM((CH,), jnp.int32),
        pltpu.VMEM((CH,), jnp.int32),
        pltpu.VMEM((CH,), jnp.int32),
        pltpu.VMEM((CH, 256), jnp.float32),
        pltpu.VMEM((CH, 256), jnp.float32),
        pltpu.VMEM((CH * 16,), jnp.float32),
        pltpu.VMEM((CH * 16,), jnp.float32),
        pltpu.SemaphoreType.DMA,
        pltpu.SemaphoreType.DMA,
        pltpu.SemaphoreType.DMA,
        pltpu.SemaphoreType.DMA,
    ],
)
def _agg_kernel(x_h, shf_h, ss_h, sr_h, bo_h, agg_h,
                bov, acc, ssv0, ssv1, srvb0, srvb1, xg0, xg1, shg0, shg1,
                sem_s0, sem_s1, sem_x0, sem_x1):
    wid = _wid()
    pltpu.sync_copy(bo_h.at[wid], bov)
    bvec = bov[...]
    zero16 = jnp.zeros((16,), jnp.float32)
    srvb = (srvb0, srvb1)
    ssv = (ssv0, ssv1)
    xg = (xg0, xg1)
    shg = (shg0, shg1)
    sem_s = (sem_s0, sem_s1)
    sem_x = (sem_x0, sem_x1)

    def block(bi, carry0):
        blk = wid * 2 + bi

        def zr(i, carry):
            acc[pl.ds(i * 16, 16)] = zero16
            return carry
        lax.fori_loop(0, BN * WAGG // 16, zr, 0)

        is0 = bi == 0
        e0 = jnp.where(is0, bvec[0], bvec[1])
        e1 = jnp.where(is0, bvec[1], bvec[2])
        astart = (e0 // 8) * 8
        nch = (e1 - astart + CH - 1) // CH
        nbase = blk * BN

        def issue_shx(c, s):
            @pl.when(c < nch)
            def _():
                cbase = astart + c * CH
                pltpu.async_copy(ss_h.at[pl.ds(cbase, CH)], ssv[s], sem_s[s])
                pltpu.async_copy(sr_h.at[pl.ds(cbase, CH)], srvb[s], sem_s[s])
                pltpu.async_copy(shf_h.at[pl.ds(cbase * 16, CH * 16)],
                                 shg[s], sem_s[s])

        def finish_shx_issue_gather(c, s):
            @pl.when(c < nch)
            def _():
                cbase = astart + c * CH
                pltpu.make_async_copy(ss_h.at[pl.ds(cbase, CH)],
                                      ssv[s], sem_s[s]).wait()
                pltpu.make_async_copy(sr_h.at[pl.ds(cbase, CH)],
                                      srvb[s], sem_s[s]).wait()
                pltpu.make_async_copy(shf_h.at[pl.ds(cbase * 16, CH * 16)],
                                      shg[s], sem_s[s]).wait()
                pltpu.async_copy(x_h.at[ssv[s]], xg[s], sem_x[s])

        def flush(cur, accs):
            rb0 = cur * WAGG
            for k in range(28):
                plsc.addupdate(acc.at[pl.ds(rb0 + k * 16, 16)], accs[k])

        def process(c, s):
            @pl.when(c < nch)
            def _():
                cbase = astart + c * CH
                pltpu.make_async_copy(x_h.at[ssv[s]], xg[s], sem_x[s]).wait()

                def group(g, carry2):
                    cur = carry2[0]
                    accs = carry2[1:]
                    gb = g * 16
                    rv16 = srvb[s][pl.ds(gb, 16)]
                    for es in range(16):
                        e = gb + es
                        ge = cbase + e
                        valid = (ge >= e0) & (ge < e1)
                        row = rv16[es] - nbase
                        row = jnp.clip(row, 0, BN - 1)

                        def do_flush(cur=cur, accs=accs):
                            flush(cur, accs)
                            return (zero16,) * 28

                        def keep(accs=accs):
                            return accs

                        accs = lax.cond(row != cur, do_flush, keep)
                        cur = row
                        bvf = jnp.broadcast_to(
                            jnp.where(valid, jnp.float32(1.0),
                                      jnp.float32(0.0)), (16,))
                        shv = shg[s][pl.ds(e * 16, 16)] * bvf
                        x16 = xg[s][e, pl.ds(0, 16)]
                        new = []
                        for k in range(12):
                            new.append(accs[k]
                                       + xg[s][e, pl.ds(k * 16, 16)] * bvf)
                        for i in range(16):
                            new.append(accs[12 + i]
                                       + jnp.broadcast_to(x16[i], (16,))
                                       * shv)
                        accs = tuple(new)
                    return (cur, *accs)

                init = (jnp.int32(0),) + (zero16,) * 28
                fin = lax.fori_loop(0, CH // 16, group, init)
                flush(fin[0], fin[1:])

        issue_shx(0, 0)
        finish_shx_issue_gather(0, 0)
        issue_shx(1, 1)

        def pair(p, carry):
            c0 = 2 * p
            c1 = c0 + 1
            finish_shx_issue_gather(c1, 1)
            process(c0, 0)
            issue_shx(c0 + 2, 0)
            process(c1, 1)
            finish_shx_issue_gather(c0 + 2, 0)
            issue_shx(c1 + 2, 1)
            return carry
        lax.fori_loop(0, (nch + 1) // 2, pair, 0)
        pltpu.sync_copy(acc, agg_h.at[pl.ds(blk * BN * WAGG, BN * WAGG)])
        return carry0

    lax.fori_loop(0, 2, block, 0)


def _t1_body(agg_ref, wp_ref, wo_ref, wsc_ref, out_ref):
    h = jax.nn.gelu(jnp.dot(agg_ref[...] / 1.5, wp_ref[...],
                            preferred_element_type=jnp.float32))
    h = jnp.dot(h, wo_ref[...], preferred_element_type=jnp.float32)
    h = h + wsc_ref[...]
    out_ref[...] = jnp.concatenate(
        [h, jnp.zeros((h.shape[0], 256 - HID), jnp.float32)], axis=1)


def _t2_body(agg_ref, x_ref, wp_ref, wo_ref, wsc_ref, out_ref):
    h = jax.nn.gelu(jnp.dot(agg_ref[...] / 1.5, wp_ref[...],
                            preferred_element_type=jnp.float32))
    h = jnp.dot(h, wo_ref[...], preferred_element_type=jnp.float32)
    h = h + jnp.dot(x_ref[...], wsc_ref[...],
                    preferred_element_type=jnp.float32)
    out_ref[...] = jnp.concatenate(
        [h, jnp.zeros((h.shape[0], 256 - HID), jnp.float32)], axis=1)


def _t3_body(agg_ref, x_ref, gi_ref, wp_ref, wo_ref, wsc_ref, out_ref, pred):
    g = pl.program_id(0)

    @pl.when(g == 0)
    def _():
        pred[...] = jnp.zeros_like(pred)

    h = jax.nn.gelu(jnp.dot(agg_ref[...] / 1.5, wp_ref[...],
                            preferred_element_type=jnp.float32))
    h = jnp.dot(h, wo_ref[...], preferred_element_type=jnp.float32)
    x3 = h + jnp.dot(x_ref[...], wsc_ref[...],
                     preferred_element_type=jnp.float32)
    gi = gi_ref[0, 0, :]
    oh = (lax.broadcasted_iota(jnp.int32, (NG, 1000), 0)
          == gi[None, :]).astype(jnp.float32)
    pred[...] += jnp.dot(oh, x3, preferred_element_type=jnp.float32)

    @pl.when(g == 9)
    def _():
        p = pred[...]
        oe = p[:, 0:1] * p[:, 1:2]
        out_ref[...] = jnp.concatenate([oe, -oe, p[:, 2:8]], axis=1)


def _full(i, j):
    return pl.BlockSpec(j, lambda g: tuple(0 for _ in j)) if i is None else None


def _reshuffle(Wp):
    a = Wp[:HID]
    b = Wp[HID:].reshape(16, 15, -1)
    b = jnp.pad(b, ((0, 0), (0, 1), (0, 0)))
    return jnp.concatenate([a, b.reshape(256, -1)], axis=0)


def kernel(positions, senders, receivers, graph_ids,
           W_pre1, W_post1, W_sc1, W_pre2, W_post2, W_sc2,
           W_pre3, W_post3, W_sc3):
    senders = senders.astype(jnp.int32)
    receivers = receivers.astype(jnp.int32)
    graph_ids = graph_ids.astype(jnp.int32)
    order = jnp.argsort(receivers)
    ss = senders[order]
    sr = receivers[order]
    ss_p = jnp.pad(ss, (0, EPAD - E))
    sr_p = jnp.pad(sr, (0, EPAD - E))
    bo = jnp.searchsorted(sr, jnp.arange(NBLK + 1, dtype=jnp.int32) * BN)
    bo = bo.astype(jnp.int32)
    bo2 = jnp.stack([bo[0:64:2], bo[1:64:2], bo[2:65:2]], axis=1)
    bo2 = jnp.pad(bo2, ((0, 0), (0, 13)))
    px = positions[:, 0]
    py = positions[:, 1]
    pz = positions[:, 2]

    shf = _sh_kernel(px, py, pz, ss_p, sr_p)
    agg1 = _agg1_kernel(shf, sr_p, bo2).reshape(NPAD, 16)

    Wp1p = jnp.concatenate([W_pre1[1:16], W_pre1[0:1]], axis=0)
    x1 = pl.pallas_call(
        _t1_body, grid=(10,),
        in_specs=[pl.BlockSpec((1000, 16), lambda g: (g, 0)),
                  pl.BlockSpec((16, HID), lambda g: (0, 0)),
                  pl.BlockSpec((HID, HID), lambda g: (0, 0)),
                  pl.BlockSpec((1, HID), lambda g: (0, 0))],
        out_specs=pl.BlockSpec((1000, 256), lambda g: (g, 0)),
        out_shape=jax.ShapeDtypeStruct((N, 256), jnp.float32),
    )(agg1, Wp1p, W_post1, W_sc1)

    agg2 = _agg_kernel(x1, shf, ss_p, sr_p, bo2).reshape(NPAD, WAGG)
    x2 = pl.pallas_call(
        _t2_body, grid=(10,),
        in_specs=[pl.BlockSpec((1000, WAGG), lambda g: (g, 0)),
                  pl.BlockSpec((1000, 256), lambda g: (g, 0)),
                  pl.BlockSpec((WAGG, HID), lambda g: (0, 0)),
                  pl.BlockSpec((HID, HID), lambda g: (0, 0)),
                  pl.BlockSpec((256, HID), lambda g: (0, 0))],
        out_specs=pl.BlockSpec((1000, 256), lambda g: (g, 0)),
        out_shape=jax.ShapeDtypeStruct((N, 256), jnp.float32),
    )(agg2, x1, _reshuffle(W_pre2), W_post2,
      jnp.pad(W_sc2, ((0, 256 - HID), (0, 0))))

    agg3 = _agg_kernel(x2, shf, ss_p, sr_p, bo2).reshape(NPAD, WAGG)
    gi3 = graph_ids.reshape(10, 1, 1000)
    logits = pl.pallas_call(
        _t3_body, grid=(10,),
        in_specs=[pl.BlockSpec((1000, WAGG), lambda g: (g, 0)),
                  pl.BlockSpec((1000, 256), lambda g: (g, 0)),
                  pl.BlockSpec((1, 1, 1000), lambda g: (g, 0, 0)),
                  pl.BlockSpec((WAGG, 8), lambda g: (0, 0)),
                  pl.BlockSpec((8, 8), lambda g: (0, 0)),
                  pl.BlockSpec((256, 8), lambda g: (0, 0))],
        out_specs=pl.BlockSpec((NG, 8), lambda g: (0, 0)),
        out_shape=jax.ShapeDtypeStruct((NG, 8), jnp.float32),
        scratch_shapes=[pltpu.VMEM((NG, 8), jnp.float32)],
    )(agg3, x2, gi3, _reshuffle(W_pre3), W_post3,
      jnp.pad(W_sc3, ((0, 256 - HID), (0, 0))))
    return logits
